# baseline (speedup 1.0000x reference)
import functools

import jax
import jax.numpy as jnp
from jax import lax
from jax.experimental import pallas as pl
from jax.experimental.pallas import tpu as pltpu
from jax.experimental.pallas import tpu_sc as plsc

N = 10000
E = 160000
DIN = 128
H = 256
ED = 16
DOUT = 128
XC = DIN + H
HID2 = 2 * H
CW = 128

NC = 2
NS = 16
LANES = 16
EHALF = E // NC
PER_S = EHALF // NS
EB = 40
NCH = PER_S // EB
ROWS_S = 624
ROWS_LAST = N - (NS - 1) * ROWS_S
OFF_LAST = (NS - 1) * ROWS_S

_HIGH = jax.lax.Precision.HIGHEST


def _dot(a, b):
    return jax.lax.dot_general(a, b, (((1,), (0,)), ((), ())),
                               precision=_HIGH,
                               preferred_element_type=jnp.float32)


def _mlp_in_body(x_ref, w_ref, b_ref, h_ref, t_ref):
    h = _dot(x_ref[...], w_ref[...]) + b_ref[...]
    h_ref[...] = h
    t_ref[0] = h[:, :CW]
    t_ref[1] = h[:, CW:]


def _mlp_in(x, W_in, b_in, blk=1000):
    grid = (N // blk,)
    return pl.pallas_call(
        _mlp_in_body,
        grid=grid,
        in_specs=[
            pl.BlockSpec((blk, DIN), lambda i: (i, 0)),
            pl.BlockSpec((DIN, H), lambda i: (0, 0)),
            pl.BlockSpec((1, H), lambda i: (0, 0)),
        ],
        out_specs=[
            pl.BlockSpec((blk, H), lambda i: (i, 0)),
            pl.BlockSpec((2, blk, CW), lambda i: (0, i, 0)),
        ],
        out_shape=[
            jax.ShapeDtypeStruct((N, H), jnp.float32),
            jax.ShapeDtypeStruct((2, N, CW), jnp.float32),
        ],
    )(x, W_in, b_in.reshape(1, H))


def _edge_feat_body(ea_ref, w_ref, b_ref, out_ref):
    ea = ea_ref[...]
    for l in range(2):
        for ch in range(3):
            col = l * XC + ch * CW
            out_ref[l, ch] = (_dot(ea, w_ref[:, col:col + CW])
                              + b_ref[:, col:col + CW])


def _edge_feat(edge_attr, Wcat, bcat, blk=2000):
    grid = (E // blk,)
    return pl.pallas_call(
        _edge_feat_body,
        grid=grid,
        in_specs=[
            pl.BlockSpec((blk, ED), lambda i: (i, 0)),
            pl.BlockSpec((ED, 2 * XC), lambda i: (0, 0)),
            pl.BlockSpec((1, 2 * XC), lambda i: (0, 0)),
        ],
        out_specs=pl.BlockSpec((2, 3, blk, CW), lambda i: (0, 0, i, 0)),
        out_shape=jax.ShapeDtypeStruct((2, 3, E, CW), jnp.float32),
    )(edge_attr, Wcat, bcat.reshape(1, 2 * XC))


def _ranged_copy(s, mk_src, mk_dst):
    @pl.when(s < NS - 1)
    def _():
        pltpu.sync_copy(mk_src(s * ROWS_S, ROWS_S), mk_dst(s * ROWS_S, ROWS_S))

    @pl.when(s == NS - 1)
    def _():
        pltpu.sync_copy(mk_src(OFF_LAST, ROWS_LAST), mk_dst(OFF_LAST, ROWS_LAST))


def _sc_body(tx_hbm, th_hbm, e_hbm, src2_hbm, dst_hbm, z_hbm, out_hbm,
             srcb, dstb, ebuf, gbuf, acc, sem):
    c = lax.axis_index("c")
    s = lax.axis_index("s")

    for ch in range(3):
        tbl = tx_hbm if ch == 0 else th_hbm
        trow0 = 0 if ch < 2 else N

        @pl.when(c == 0)
        def _():
            _ranged_copy(s, lambda o, n: tbl.at[pl.ds(trow0 + o, n)],
                         lambda o, n: acc.at[pl.ds(o, n)])

        @pl.when(c == 1)
        def _():
            _ranged_copy(s, lambda o, n: z_hbm.at[pl.ds(o, n)],
                         lambda o, n: acc.at[pl.ds(o, n)])

        plsc.subcore_barrier()

        idx0 = 0 if ch < 2 else E

        def chunk(k, carry):
            base = c * EHALF + s * PER_S + k * EB
            pltpu.sync_copy(src2_hbm.at[pl.ds(idx0 + base, EB)], srcb)
            pltpu.sync_copy(dst_hbm.at[pl.ds(base, EB)], dstb)
            pltpu.sync_copy(e_hbm.at[pl.ds(ch * E + base, EB)], ebuf)
            pltpu.async_copy(tbl.at[srcb], gbuf, sem).wait()

            def row(i, carry2):
                for j in range(CW // LANES):
                    sl = pl.ds(j * LANES, LANES)
                    v = gbuf[i, sl] + ebuf[i, sl]
                    gbuf[i, sl] = jnp.maximum(v, 0.0)
                return carry2

            lax.fori_loop(0, EB, row, 0, unroll=2)
            pltpu.sync_copy(gbuf, acc.at[dstb], add=True)
            return carry

        lax.fori_loop(0, NCH, chunk, 0)
        plsc.subcore_barrier()

        _ranged_copy(s, lambda o, n: acc.at[pl.ds(o, n)],
                     lambda o, n: out_hbm.at[c, ch, pl.ds(o, n)])
        plsc.subcore_barrier()


@functools.partial(
    pl.kernel,
    out_type=jax.ShapeDtypeStruct((NC, 3, N, CW), jnp.float32),
    mesh=plsc.VectorSubcoreMesh(core_axis_name="c", subcore_axis_name="s",
                                num_cores=NC, num_subcores=NS),
    scratch_types=[
        pltpu.VMEM((EB,), jnp.int32),
        pltpu.VMEM((EB,), jnp.int32),
        pltpu.VMEM((EB, CW), jnp.float32),
        pltpu.VMEM((EB, CW), jnp.float32),
        pltpu.VMEM_SHARED((N, CW), jnp.float32),
        pltpu.SemaphoreType.DMA,
    ],
)
def _sc_aggregate(tx_hbm, th_hbm, e_hbm, src2_hbm, dst_hbm, z_hbm, out_hbm,
                  srcb, dstb, ebuf, gbuf, acc, sem):
    _sc_body(tx_hbm, th_hbm, e_hbm, src2_hbm, dst_hbm, z_hbm, out_hbm,
             srcb, dstb, ebuf, gbuf, acc, sem)


def _node_mlp(pre_refs, w1_refs, p):
    h = p['b1']
    for ch in range(3):
        pre = pre_refs[ch][...] + pre_refs[3 + ch][...]
        h = h + _dot(pre, w1_refs[ch][...])
    mu = jnp.mean(h, axis=-1, keepdims=True)
    var = jnp.mean((h - mu) ** 2, axis=-1, keepdims=True)
    h = p['g'] * (h - mu) / jnp.sqrt(var + 1e-5) + p['bt']
    h = jnp.where(h >= 0, h, 0.1 * h)
    h = _dot(h, p['W2']) + p['b2']
    h = jnp.where(h >= 0, h, 0.1 * h)
    return _dot(h, p['W3']) + p['b3']


def _mlp0_body(p00, p01, p02, p10, p11, p12,
               w1a, w1b, w1c, b1, g, bt, w2, b2, w3, b3,
               h_ref, t_ref):
    p = dict(b1=b1[...], g=g[...], bt=bt[...], W2=w2[...], b2=b2[...],
             W3=w3[...], b3=b3[...])
    h = _node_mlp((p00, p01, p02, p10, p11, p12), (w1a, w1b, w1c), p)
    h_ref[...] = h
    t_ref[0] = h[:, :CW]
    t_ref[1] = h[:, CW:]


def _pre_specs(blk):
    return [pl.BlockSpec((blk, CW), lambda i: (i, 0)) for _ in range(6)]


def _wspec(shp):
    return pl.BlockSpec(shp, lambda i: (0, 0))


def _mlp_layer0(pre, W1, b1, g, bt, W2, b2, W3, b3, blk=1000):
    grid = (N // blk,)
    return pl.pallas_call(
        _mlp0_body,
        grid=grid,
        in_specs=_pre_specs(blk) + [
            _wspec((CW, HID2)), _wspec((CW, HID2)), _wspec((CW, HID2)),
            _wspec((1, HID2)), _wspec((1, HID2)), _wspec((1, HID2)),
            _wspec((HID2, HID2)), _wspec((1, HID2)),
            _wspec((HID2, H)), _wspec((1, H)),
        ],
        out_specs=[
            pl.BlockSpec((blk, H), lambda i: (i, 0)),
            pl.BlockSpec((2, blk, CW), lambda i: (0, i, 0)),
        ],
        out_shape=[
            jax.ShapeDtypeStruct((N, H), jnp.float32),
            jax.ShapeDtypeStruct((2, N, CW), jnp.float32),
        ],
    )(pre[0, 0], pre[0, 1], pre[0, 2], pre[1, 0], pre[1, 1], pre[1, 2],
      W1[:CW], W1[CW:2 * CW], W1[2 * CW:], b1.reshape(1, HID2),
      g.reshape(1, HID2), bt.reshape(1, HID2), W2, b2.reshape(1, HID2),
      W3, b3.reshape(1, H))


def _mlp1_body(p00, p01, p02, p10, p11, p12, h0_ref, h1_ref,
               w1a, w1b, w1c, b1, g, bt, w2, b2, w3, b3,
               wo0, wo1, wo2, bo, out_ref):
    p = dict(b1=b1[...], g=g[...], bt=bt[...], W2=w2[...], b2=b2[...],
             W3=w3[...], b3=b3[...])
    h2 = _node_mlp((p00, p01, p02, p10, p11, p12), (w1a, w1b, w1c), p)
    out_ref[...] = (_dot(h0_ref[...], wo0[...]) + _dot(h1_ref[...], wo1[...])
                    + _dot(h2, wo2[...]) + bo[...])


def _mlp_layer1_out(pre, h0, h1, W1, b1, g, bt, W2, b2, W3, b3,
                    W_out, b_out, blk=1000):
    grid = (N // blk,)
    return pl.pallas_call(
        _mlp1_body,
        grid=grid,
        in_specs=_pre_specs(blk) + [
            pl.BlockSpec((blk, H), lambda i: (i, 0)),
            pl.BlockSpec((blk, H), lambda i: (i, 0)),
            _wspec((CW, HID2)), _wspec((CW, HID2)), _wspec((CW, HID2)),
            _wspec((1, HID2)), _wspec((1, HID2)), _wspec((1, HID2)),
            _wspec((HID2, HID2)), _wspec((1, HID2)),
            _wspec((HID2, H)), _wspec((1, H)),
            _wspec((H, DOUT)), _wspec((H, DOUT)), _wspec((H, DOUT)),
            _wspec((1, DOUT)),
        ],
        out_specs=pl.BlockSpec((blk, DOUT), lambda i: (i, 0)),
        out_shape=jax.ShapeDtypeStruct((N, DOUT), jnp.float32),
    )(pre[0, 0], pre[0, 1], pre[0, 2], pre[1, 0], pre[1, 1], pre[1, 2],
      h0, h1,
      W1[:CW], W1[CW:2 * CW], W1[2 * CW:], b1.reshape(1, HID2),
      g.reshape(1, HID2), bt.reshape(1, HID2), W2, b2.reshape(1, HID2),
      W3, b3.reshape(1, H),
      W_out[:H], W_out[H:2 * H], W_out[2 * H:], b_out.reshape(1, DOUT))


def kernel(x, edge_index, edge_attr, W_in, b_in,
           We0, be0, W1_0, b1_0, g_0, bt_0, W2_0, b2_0, W3_0, b3_0,
           We1, be1, W1_1, b1_1, g_1, bt_1, W2_1, b2_1, W3_1, b3_1,
           W_out, b_out):
    src = edge_index[0]
    dst = edge_index[1]
    src2 = jnp.concatenate([src, src + N])
    zeros = jnp.zeros((N, CW), jnp.float32)

    h0, th0 = _mlp_in(x, W_in, b_in)

    Wcat = jnp.concatenate([We0, We1], axis=1)
    bcat = jnp.concatenate([be0, be1])
    e_all = _edge_feat(edge_attr, Wcat, bcat)

    pre0 = _sc_aggregate(x, th0.reshape(2 * N, CW),
                         e_all[0].reshape(3 * E, CW), src2, dst, zeros)
    h1, th1 = _mlp_layer0(pre0, W1_0, b1_0, g_0, bt_0, W2_0, b2_0, W3_0, b3_0)

    pre1 = _sc_aggregate(x, th1.reshape(2 * N, CW),
                         e_all[1].reshape(3 * E, CW), src2, dst, zeros)
    out = _mlp_layer1_out(pre1, h0, h1,
                          W1_1, b1_1, g_1, bt_1, W2_1, b2_1, W3_1, b3_1,
                          W_out, b_out)
    return out


# speedup vs baseline: 1.6933x; 1.6933x over previous
import functools

import jax
import jax.numpy as jnp
from jax import lax
from jax.experimental import pallas as pl
from jax.experimental.pallas import tpu as pltpu
from jax.experimental.pallas import tpu_sc as plsc

N = 10000
E = 160000
DIN = 128
H = 256
ED = 16
DOUT = 128
XC = DIN + H
HID2 = 2 * H
CW = 128

NC = 2
NS = 16
LANES = 16
EHALF = E // NC
PER_S = EHALF // NS
EB = 40
NCH = PER_S // EB
ROWS_S = 624
ROWS_LAST = N - (NS - 1) * ROWS_S
OFF_LAST = (NS - 1) * ROWS_S

_HIGH = jax.lax.Precision.HIGHEST


def _dot(a, b):
    return jax.lax.dot_general(a, b, (((1,), (0,)), ((), ())),
                               precision=_HIGH,
                               preferred_element_type=jnp.float32)


def _mlp_in_body(x_ref, w_ref, b_ref, h_ref, t_ref):
    h = _dot(x_ref[...], w_ref[...]) + b_ref[...]
    h_ref[...] = h
    t_ref[0] = h[:, :CW]
    t_ref[1] = h[:, CW:]


def _mlp_in(x, W_in, b_in, blk=1000):
    grid = (N // blk,)
    return pl.pallas_call(
        _mlp_in_body,
        grid=grid,
        in_specs=[
            pl.BlockSpec((blk, DIN), lambda i: (i, 0)),
            pl.BlockSpec((DIN, H), lambda i: (0, 0)),
            pl.BlockSpec((1, H), lambda i: (0, 0)),
        ],
        out_specs=[
            pl.BlockSpec((blk, H), lambda i: (i, 0)),
            pl.BlockSpec((2, blk, CW), lambda i: (0, i, 0)),
        ],
        out_shape=[
            jax.ShapeDtypeStruct((N, H), jnp.float32),
            jax.ShapeDtypeStruct((2, N, CW), jnp.float32),
        ],
    )(x, W_in, b_in.reshape(1, H))


def _edge_feat_body(ea_ref, w_ref, b_ref, out_ref):
    ea = ea_ref[...]
    for l in range(2):
        for ch in range(3):
            col = l * XC + ch * CW
            out_ref[l, ch] = (_dot(ea, w_ref[:, col:col + CW])
                              + b_ref[:, col:col + CW])


def _edge_feat(edge_attr, Wcat, bcat, blk=2000):
    grid = (E // blk,)
    return pl.pallas_call(
        _edge_feat_body,
        grid=grid,
        in_specs=[
            pl.BlockSpec((blk, ED), lambda i: (i, 0)),
            pl.BlockSpec((ED, 2 * XC), lambda i: (0, 0)),
            pl.BlockSpec((1, 2 * XC), lambda i: (0, 0)),
        ],
        out_specs=pl.BlockSpec((2, 3, blk, CW), lambda i: (0, 0, i, 0)),
        out_shape=jax.ShapeDtypeStruct((2, 3, E, CW), jnp.float32),
    )(edge_attr, Wcat, bcat.reshape(1, 2 * XC))


def _ranged_copy(s, mk_src, mk_dst):
    @pl.when(s < NS - 1)
    def _():
        pltpu.sync_copy(mk_src(s * ROWS_S, ROWS_S), mk_dst(s * ROWS_S, ROWS_S))

    @pl.when(s == NS - 1)
    def _():
        pltpu.sync_copy(mk_src(OFF_LAST, ROWS_LAST), mk_dst(OFF_LAST, ROWS_LAST))


def _sc_body(tx_hbm, th_hbm, e_hbm, src2_hbm, dst_hbm, z_hbm, out_hbm,
             srcall, dstb, ebuf, gbuf, acc,
             semE0, semE1, semG0, semG1, semD0, semD1):
    c = lax.axis_index("c")
    s = lax.axis_index("s")
    semE = (semE0, semE1)
    semG = (semG0, semG1)
    semD = (semD0, semD1)

    for ch in range(3):
        tbl = tx_hbm if ch == 0 else th_hbm
        trow0 = 0 if ch < 2 else N

        idx0 = (0 if ch < 2 else E) + c * EHALF + s * PER_S
        pltpu.sync_copy(src2_hbm.at[pl.ds(idx0, PER_S)], srcall)
        dst0 = c * EHALF + s * PER_S

        @pl.when(c == 0)
        def _():
            _ranged_copy(s, lambda o, n: tbl.at[pl.ds(trow0 + o, n)],
                         lambda o, n: acc.at[pl.ds(o, n)])

        @pl.when(c == 1)
        def _():
            _ranged_copy(s, lambda o, n: z_hbm.at[pl.ds(o, n)],
                         lambda o, n: acc.at[pl.ds(o, n)])

        plsc.subcore_barrier()

        def e_slice(k):
            base = ch * E + c * EHALF + s * PER_S + k * EB
            return e_hbm.at[pl.ds(base, EB)]

        def src_idx(k):
            return srcall.at[pl.ds(k * EB, EB)]

        def fetch(k, b):
            pltpu.async_copy(e_slice(k), ebuf.at[b], semE[b])
            pltpu.async_copy(dst_hbm.at[pl.ds(dst0 + k * EB, EB)],
                             dstb.at[b], semD[b])
            pltpu.async_copy(tbl.at[src_idx(k)], gbuf.at[b], semG[b])

        def wait_fetch(k, b):
            pltpu.make_async_copy(e_slice(k), ebuf.at[b], semE[b]).wait()
            pltpu.make_async_copy(dst_hbm.at[pl.ds(dst0 + k * EB, EB)],
                                  dstb.at[b], semD[b]).wait()
            pltpu.make_async_copy(tbl.at[src_idx(k)], gbuf.at[b],
                                  semG[b]).wait()

        def compute_scatter(k, b):
            def row(i, carry2):
                for j in range(CW // LANES):
                    sl = pl.ds(j * LANES, LANES)
                    v = gbuf[b, i, sl] + ebuf[b, i, sl]
                    gbuf[b, i, sl] = jnp.maximum(v, 0.0)
                return carry2

            lax.fori_loop(0, EB, row, 0, unroll=2)
            pltpu.sync_copy(gbuf.at[b], acc.at[dstb.at[b]], add=True)

        fetch(0, 0)

        def pair(g, carry):
            k0 = 2 * g
            fetch(k0 + 1, 1)
            wait_fetch(k0, 0)
            compute_scatter(k0, 0)
            fetch(k0 + 2, 0)
            wait_fetch(k0 + 1, 1)
            compute_scatter(k0 + 1, 1)
            return carry

        lax.fori_loop(0, (NCH - 1) // 2, pair, 0)
        wait_fetch(NCH - 1, 0)
        compute_scatter(NCH - 1, 0)

        plsc.subcore_barrier()

        _ranged_copy(s, lambda o, n: acc.at[pl.ds(o, n)],
                     lambda o, n: out_hbm.at[c, ch, pl.ds(o, n)])
        plsc.subcore_barrier()


@functools.partial(
    pl.kernel,
    out_type=jax.ShapeDtypeStruct((NC, 3, N, CW), jnp.float32),
    mesh=plsc.VectorSubcoreMesh(core_axis_name="c", subcore_axis_name="s",
                                num_cores=NC, num_subcores=NS),
    scratch_types=[
        pltpu.VMEM((PER_S,), jnp.int32),
        pltpu.VMEM((2, EB), jnp.int32),
        pltpu.VMEM((2, EB, CW), jnp.float32),
        pltpu.VMEM((2, EB, CW), jnp.float32),
        pltpu.VMEM_SHARED((N, CW), jnp.float32),
        pltpu.SemaphoreType.DMA,
        pltpu.SemaphoreType.DMA,
        pltpu.SemaphoreType.DMA,
        pltpu.SemaphoreType.DMA,
        pltpu.SemaphoreType.DMA,
        pltpu.SemaphoreType.DMA,
    ],
)
def _sc_aggregate(tx_hbm, th_hbm, e_hbm, src2_hbm, dst_hbm, z_hbm, out_hbm,
                  srcall, dstb, ebuf, gbuf, acc,
                  semE0, semE1, semG0, semG1, semD0, semD1):
    _sc_body(tx_hbm, th_hbm, e_hbm, src2_hbm, dst_hbm, z_hbm, out_hbm,
             srcall, dstb, ebuf, gbuf, acc,
             semE0, semE1, semG0, semG1, semD0, semD1)


def _node_mlp(pre_refs, w1_refs, p):
    h = p['b1']
    for ch in range(3):
        pre = pre_refs[ch][...] + pre_refs[3 + ch][...]
        h = h + _dot(pre, w1_refs[ch][...])
    mu = jnp.mean(h, axis=-1, keepdims=True)
    var = jnp.mean((h - mu) ** 2, axis=-1, keepdims=True)
    h = p['g'] * (h - mu) / jnp.sqrt(var + 1e-5) + p['bt']
    h = jnp.where(h >= 0, h, 0.1 * h)
    h = _dot(h, p['W2']) + p['b2']
    h = jnp.where(h >= 0, h, 0.1 * h)
    return _dot(h, p['W3']) + p['b3']


def _mlp0_body(p00, p01, p02, p10, p11, p12,
               w1a, w1b, w1c, b1, g, bt, w2, b2, w3, b3,
               h_ref, t_ref):
    p = dict(b1=b1[...], g=g[...], bt=bt[...], W2=w2[...], b2=b2[...],
             W3=w3[...], b3=b3[...])
    h = _node_mlp((p00, p01, p02, p10, p11, p12), (w1a, w1b, w1c), p)
    h_ref[...] = h
    t_ref[0] = h[:, :CW]
    t_ref[1] = h[:, CW:]


def _pre_specs(blk):
    return [pl.BlockSpec((blk, CW), lambda i: (i, 0)) for _ in range(6)]


def _wspec(shp):
    return pl.BlockSpec(shp, lambda i: (0, 0))


def _mlp_layer0(pre, W1, b1, g, bt, W2, b2, W3, b3, blk=1000):
    grid = (N // blk,)
    return pl.pallas_call(
        _mlp0_body,
        grid=grid,
        in_specs=_pre_specs(blk) + [
            _wspec((CW, HID2)), _wspec((CW, HID2)), _wspec((CW, HID2)),
            _wspec((1, HID2)), _wspec((1, HID2)), _wspec((1, HID2)),
            _wspec((HID2, HID2)), _wspec((1, HID2)),
            _wspec((HID2, H)), _wspec((1, H)),
        ],
        out_specs=[
            pl.BlockSpec((blk, H), lambda i: (i, 0)),
            pl.BlockSpec((2, blk, CW), lambda i: (0, i, 0)),
        ],
        out_shape=[
            jax.ShapeDtypeStruct((N, H), jnp.float32),
            jax.ShapeDtypeStruct((2, N, CW), jnp.float32),
        ],
    )(pre[0, 0], pre[0, 1], pre[0, 2], pre[1, 0], pre[1, 1], pre[1, 2],
      W1[:CW], W1[CW:2 * CW], W1[2 * CW:], b1.reshape(1, HID2),
      g.reshape(1, HID2), bt.reshape(1, HID2), W2, b2.reshape(1, HID2),
      W3, b3.reshape(1, H))


def _mlp1_body(p00, p01, p02, p10, p11, p12, h0_ref, h1_ref,
               w1a, w1b, w1c, b1, g, bt, w2, b2, w3, b3,
               wo0, wo1, wo2, bo, out_ref):
    p = dict(b1=b1[...], g=g[...], bt=bt[...], W2=w2[...], b2=b2[...],
             W3=w3[...], b3=b3[...])
    h2 = _node_mlp((p00, p01, p02, p10, p11, p12), (w1a, w1b, w1c), p)
    out_ref[...] = (_dot(h0_ref[...], wo0[...]) + _dot(h1_ref[...], wo1[...])
                    + _dot(h2, wo2[...]) + bo[...])


def _mlp_layer1_out(pre, h0, h1, W1, b1, g, bt, W2, b2, W3, b3,
                    W_out, b_out, blk=1000):
    grid = (N // blk,)
    return pl.pallas_call(
        _mlp1_body,
        grid=grid,
        in_specs=_pre_specs(blk) + [
            pl.BlockSpec((blk, H), lambda i: (i, 0)),
            pl.BlockSpec((blk, H), lambda i: (i, 0)),
            _wspec((CW, HID2)), _wspec((CW, HID2)), _wspec((CW, HID2)),
            _wspec((1, HID2)), _wspec((1, HID2)), _wspec((1, HID2)),
            _wspec((HID2, HID2)), _wspec((1, HID2)),
            _wspec((HID2, H)), _wspec((1, H)),
            _wspec((H, DOUT)), _wspec((H, DOUT)), _wspec((H, DOUT)),
            _wspec((1, DOUT)),
        ],
        out_specs=pl.BlockSpec((blk, DOUT), lambda i: (i, 0)),
        out_shape=jax.ShapeDtypeStruct((N, DOUT), jnp.float32),
    )(pre[0, 0], pre[0, 1], pre[0, 2], pre[1, 0], pre[1, 1], pre[1, 2],
      h0, h1,
      W1[:CW], W1[CW:2 * CW], W1[2 * CW:], b1.reshape(1, HID2),
      g.reshape(1, HID2), bt.reshape(1, HID2), W2, b2.reshape(1, HID2),
      W3, b3.reshape(1, H),
      W_out[:H], W_out[H:2 * H], W_out[2 * H:], b_out.reshape(1, DOUT))


def kernel(x, edge_index, edge_attr, W_in, b_in,
           We0, be0, W1_0, b1_0, g_0, bt_0, W2_0, b2_0, W3_0, b3_0,
           We1, be1, W1_1, b1_1, g_1, bt_1, W2_1, b2_1, W3_1, b3_1,
           W_out, b_out):
    src = edge_index[0]
    dst = edge_index[1]
    src2 = jnp.concatenate([src, src + N])
    zeros = jnp.zeros((N, CW), jnp.float32)

    h0, th0 = _mlp_in(x, W_in, b_in)

    Wcat = jnp.concatenate([We0, We1], axis=1)
    bcat = jnp.concatenate([be0, be1])
    e_all = _edge_feat(edge_attr, Wcat, bcat)


    pre0 = _sc_aggregate(x, th0.reshape(2 * N, CW),
                         e_all[0].reshape(3 * E, CW), src2, dst, zeros)
    h1, th1 = _mlp_layer0(pre0, W1_0, b1_0, g_0, bt_0, W2_0, b2_0, W3_0, b3_0)

    pre1 = _sc_aggregate(x, th1.reshape(2 * N, CW),
                         e_all[1].reshape(3 * E, CW), src2, dst, zeros)
    out = _mlp_layer1_out(pre1, h0, h1,
                          W1_1, b1_1, g_1, bt_1, W2_1, b2_1, W3_1, b3_1,
                          W_out, b_out)
    return out


# speedup vs baseline: 2.2226x; 1.3126x over previous
import functools

import jax
import jax.numpy as jnp
from jax import lax
from jax.experimental import pallas as pl
from jax.experimental.pallas import tpu as pltpu
from jax.experimental.pallas import tpu_sc as plsc

N = 10000
E = 160000
DIN = 128
H = 256
ED = 16
DOUT = 128
XC = DIN + H
HID2 = 2 * H
CW = 128

NC = 2
NS = 16
NW = NC * NS
LANES = 16
EB = 48
PER_S = 5184
NCH = PER_S // EB
EREAL_S = E // NW
EP = NW * PER_S
NPAD = 8
ROWS_S = 624
ROWS_LAST = N - (NS - 1) * ROWS_S
OFF_LAST = (NS - 1) * ROWS_S

_HIGH = jax.lax.Precision.HIGHEST


def _dot(a, b):
    return jax.lax.dot_general(a, b, (((1,), (0,)), ((), ())),
                               precision=_HIGH,
                               preferred_element_type=jnp.float32)


def _mlp_in_body(x_ref, w_ref, b_ref, h_ref, t_ref):
    h = _dot(x_ref[...], w_ref[...]) + b_ref[...]
    h_ref[...] = h
    t_ref[0] = h[:, :CW]
    t_ref[1] = h[:, CW:]


def _mlp_in(x, W_in, b_in, blk=1000):
    grid = (N // blk,)
    return pl.pallas_call(
        _mlp_in_body,
        grid=grid,
        in_specs=[
            pl.BlockSpec((blk, DIN), lambda i: (i, 0)),
            pl.BlockSpec((DIN, H), lambda i: (0, 0)),
            pl.BlockSpec((1, H), lambda i: (0, 0)),
        ],
        out_specs=[
            pl.BlockSpec((blk, H), lambda i: (i, 0)),
            pl.BlockSpec((2, blk, CW), lambda i: (0, i, 0)),
        ],
        out_shape=[
            jax.ShapeDtypeStruct((N, H), jnp.float32),
            jax.ShapeDtypeStruct((2, N, CW), jnp.float32),
        ],
    )(x, W_in, b_in.reshape(1, H))


def _edge_feat_body(ea_ref, w_ref, b_ref, out_ref):
    ea = ea_ref[...]
    for l in range(2):
        for ch in range(3):
            col = l * XC + ch * CW
            out_ref[l, ch] = (_dot(ea, w_ref[:, col:col + CW])
                              + b_ref[:, col:col + CW])


def _edge_feat(edge_attr, Wcat, bcat, blk=2048):
    grid = (EP // blk,)
    return pl.pallas_call(
        _edge_feat_body,
        grid=grid,
        in_specs=[
            pl.BlockSpec((blk, ED), lambda i: (i, 0)),
            pl.BlockSpec((ED, 2 * XC), lambda i: (0, 0)),
            pl.BlockSpec((1, 2 * XC), lambda i: (0, 0)),
        ],
        out_specs=pl.BlockSpec((2, 3, blk, CW), lambda i: (0, 0, i, 0)),
        out_shape=jax.ShapeDtypeStruct((2, 3, EP, CW), jnp.float32),
    )(edge_attr, Wcat, bcat.reshape(1, 2 * XC))


def _ranged_copy(s, mk_src, mk_dst):
    @pl.when(s < NS - 1)
    def _():
        pltpu.sync_copy(mk_src(s * ROWS_S, ROWS_S), mk_dst(s * ROWS_S, ROWS_S))

    @pl.when(s == NS - 1)
    def _():
        pltpu.sync_copy(mk_src(OFF_LAST, ROWS_LAST), mk_dst(OFF_LAST, ROWS_LAST))


def _sc_body(tx_hbm, th_hbm, e_hbm, src2_hbm, dst_hbm, z_hbm, out_hbm,
             srcall, dstb, ebuf, acc, semE, semG, semD, semS):
    c = lax.axis_index("c")
    s = lax.axis_index("s")
    w = c * NS + s

    for ch in range(3):
        tbl = tx_hbm if ch == 0 else th_hbm
        trow0 = 0 if ch < 2 else N

        idx0 = (0 if ch < 2 else EP) + w * PER_S
        pltpu.sync_copy(src2_hbm.at[pl.ds(idx0, PER_S)], srcall)
        dst0 = w * PER_S

        @pl.when(c == 0)
        def _():
            _ranged_copy(s, lambda o, n: tbl.at[pl.ds(trow0 + o, n)],
                         lambda o, n: acc.at[pl.ds(o, n)])

        @pl.when(c == 1)
        def _():
            _ranged_copy(s, lambda o, n: z_hbm.at[pl.ds(o, n)],
                         lambda o, n: acc.at[pl.ds(o, n)])

        plsc.subcore_barrier()

        def e_slice(k):
            return e_hbm.at[pl.ds(ch * EP + dst0 + k * EB, EB)]

        def src_idx(k):
            return srcall.at[pl.ds(k * EB, EB)]

        def fetch_ed(k, b):
            pltpu.async_copy(e_slice(k), ebuf.at[b], semE[b])
            pltpu.async_copy(dst_hbm.at[pl.ds(dst0 + k * EB, EB)],
                             dstb.at[b], semD[b])

        def wait_e(k, b):
            pltpu.make_async_copy(e_slice(k), ebuf.at[b], semE[b]).wait()

        def gather_add(k, b):
            pltpu.async_copy(tbl.at[src_idx(k)], ebuf.at[b], semG[b],
                             add=True)

        def wait_gather(k, b):
            pltpu.make_async_copy(tbl.at[src_idx(k)], ebuf.at[b],
                                  semG[b]).wait()

        def scatter(k, b):
            pltpu.make_async_copy(dst_hbm.at[pl.ds(dst0 + k * EB, EB)],
                                  dstb.at[b], semD[b]).wait()
            pltpu.async_copy(ebuf.at[b], acc.at[dstb.at[b]], semS[b],
                             add=True)

        def wait_scatter(k, b):
            pltpu.make_async_copy(ebuf.at[b], acc.at[dstb.at[b]],
                                  semS[b]).wait()

        def relu(b):
            def row(i, carry2):
                for j in range(CW // LANES):
                    sl = pl.ds(j * LANES, LANES)
                    ebuf[b, i, sl] = jnp.maximum(ebuf[b, i, sl], 0.0)
                return carry2

            lax.fori_loop(0, EB, row, 0, unroll=2)

        fetch_ed(0, 0)
        fetch_ed(1, 1)
        wait_e(0, 0)
        gather_add(0, 0)

        def group(g, carry):
            for j in range(3):
                k = 3 * g + j
                b, b1, b2 = j, (j + 1) % 3, (j + 2) % 3

                @pl.when(k <= NCH - 2)
                def _():
                    wait_e(k + 1, b1)
                    gather_add(k + 1, b1)

                wait_gather(k, b)
                relu(b)
                scatter(k, b)

                @pl.when(k >= 1)
                def _():
                    wait_scatter(k - 1, b2)

                @pl.when(k <= NCH - 3)
                def _():
                    fetch_ed(k + 2, b2)
            return carry

        lax.fori_loop(0, NCH // 3, group, 0)
        wait_scatter(NCH - 1, (NCH - 1) % 3)

        plsc.subcore_barrier()

        _ranged_copy(s, lambda o, n: acc.at[pl.ds(o, n)],
                     lambda o, n: out_hbm.at[c, ch, pl.ds(o, n)])
        plsc.subcore_barrier()


@functools.partial(
    pl.kernel,
    out_type=jax.ShapeDtypeStruct((NC, 3, N, CW), jnp.float32),
    mesh=plsc.VectorSubcoreMesh(core_axis_name="c", subcore_axis_name="s",
                                num_cores=NC, num_subcores=NS),
    scratch_types=[
        pltpu.VMEM((PER_S,), jnp.int32),
        pltpu.VMEM((3, EB), jnp.int32),
        pltpu.VMEM((3, EB, CW), jnp.float32),
        pltpu.VMEM_SHARED((N + NPAD, CW), jnp.float32),
    ] + [pltpu.SemaphoreType.DMA] * 12,
)
def _sc_aggregate(tx_hbm, th_hbm, e_hbm, src2_hbm, dst_hbm, z_hbm, out_hbm,
                  srcall, dstb, ebuf, acc, *sems):
    _sc_body(tx_hbm, th_hbm, e_hbm, src2_hbm, dst_hbm, z_hbm, out_hbm,
             srcall, dstb, ebuf, acc,
             sems[0:3], sems[3:6], sems[6:9], sems[9:12])


def _node_mlp(pre_refs, w1_refs, p):
    h = p['b1']
    for ch in range(3):
        pre = pre_refs[ch][...] + pre_refs[3 + ch][...]
        h = h + _dot(pre, w1_refs[ch][...])
    mu = jnp.mean(h, axis=-1, keepdims=True)
    var = jnp.mean((h - mu) ** 2, axis=-1, keepdims=True)
    h = p['g'] * (h - mu) / jnp.sqrt(var + 1e-5) + p['bt']
    h = jnp.where(h >= 0, h, 0.1 * h)
    h = _dot(h, p['W2']) + p['b2']
    h = jnp.where(h >= 0, h, 0.1 * h)
    return _dot(h, p['W3']) + p['b3']


def _mlp0_body(p00, p01, p02, p10, p11, p12,
               w1a, w1b, w1c, b1, g, bt, w2, b2, w3, b3,
               h_ref, t_ref):
    p = dict(b1=b1[...], g=g[...], bt=bt[...], W2=w2[...], b2=b2[...],
             W3=w3[...], b3=b3[...])
    h = _node_mlp((p00, p01, p02, p10, p11, p12), (w1a, w1b, w1c), p)
    h_ref[...] = h
    t_ref[0] = h[:, :CW]
    t_ref[1] = h[:, CW:]


def _pre_specs(blk):
    return [pl.BlockSpec((blk, CW), lambda i: (i, 0)) for _ in range(6)]


def _wspec(shp):
    return pl.BlockSpec(shp, lambda i: (0, 0))


def _mlp_layer0(pre, W1, b1, g, bt, W2, b2, W3, b3, blk=1000):
    grid = (N // blk,)
    return pl.pallas_call(
        _mlp0_body,
        grid=grid,
        in_specs=_pre_specs(blk) + [
            _wspec((CW, HID2)), _wspec((CW, HID2)), _wspec((CW, HID2)),
            _wspec((1, HID2)), _wspec((1, HID2)), _wspec((1, HID2)),
            _wspec((HID2, HID2)), _wspec((1, HID2)),
            _wspec((HID2, H)), _wspec((1, H)),
        ],
        out_specs=[
            pl.BlockSpec((blk, H), lambda i: (i, 0)),
            pl.BlockSpec((2, blk, CW), lambda i: (0, i, 0)),
        ],
        out_shape=[
            jax.ShapeDtypeStruct((N, H), jnp.float32),
            jax.ShapeDtypeStruct((2, N, CW), jnp.float32),
        ],
    )(pre[0, 0], pre[0, 1], pre[0, 2], pre[1, 0], pre[1, 1], pre[1, 2],
      W1[:CW], W1[CW:2 * CW], W1[2 * CW:], b1.reshape(1, HID2),
      g.reshape(1, HID2), bt.reshape(1, HID2), W2, b2.reshape(1, HID2),
      W3, b3.reshape(1, H))


def _mlp1_body(p00, p01, p02, p10, p11, p12, h0_ref, h1_ref,
               w1a, w1b, w1c, b1, g, bt, w2, b2, w3, b3,
               wo0, wo1, wo2, bo, out_ref):
    p = dict(b1=b1[...], g=g[...], bt=bt[...], W2=w2[...], b2=b2[...],
             W3=w3[...], b3=b3[...])
    h2 = _node_mlp((p00, p01, p02, p10, p11, p12), (w1a, w1b, w1c), p)
    out_ref[...] = (_dot(h0_ref[...], wo0[...]) + _dot(h1_ref[...], wo1[...])
                    + _dot(h2, wo2[...]) + bo[...])


def _mlp_layer1_out(pre, h0, h1, W1, b1, g, bt, W2, b2, W3, b3,
                    W_out, b_out, blk=1000):
    grid = (N // blk,)
    return pl.pallas_call(
        _mlp1_body,
        grid=grid,
        in_specs=_pre_specs(blk) + [
            pl.BlockSpec((blk, H), lambda i: (i, 0)),
            pl.BlockSpec((blk, H), lambda i: (i, 0)),
            _wspec((CW, HID2)), _wspec((CW, HID2)), _wspec((CW, HID2)),
            _wspec((1, HID2)), _wspec((1, HID2)), _wspec((1, HID2)),
            _wspec((HID2, HID2)), _wspec((1, HID2)),
            _wspec((HID2, H)), _wspec((1, H)),
            _wspec((H, DOUT)), _wspec((H, DOUT)), _wspec((H, DOUT)),
            _wspec((1, DOUT)),
        ],
        out_specs=pl.BlockSpec((blk, DOUT), lambda i: (i, 0)),
        out_shape=jax.ShapeDtypeStruct((N, DOUT), jnp.float32),
    )(pre[0, 0], pre[0, 1], pre[0, 2], pre[1, 0], pre[1, 1], pre[1, 2],
      h0, h1,
      W1[:CW], W1[CW:2 * CW], W1[2 * CW:], b1.reshape(1, HID2),
      g.reshape(1, HID2), bt.reshape(1, HID2), W2, b2.reshape(1, HID2),
      W3, b3.reshape(1, H),
      W_out[:H], W_out[H:2 * H], W_out[2 * H:], b_out.reshape(1, DOUT))


def kernel(x, edge_index, edge_attr, W_in, b_in,
           We0, be0, W1_0, b1_0, g_0, bt_0, W2_0, b2_0, W3_0, b3_0,
           We1, be1, W1_1, b1_1, g_1, bt_1, W2_1, b2_1, W3_1, b3_1,
           W_out, b_out):
    npad = PER_S - EREAL_S
    pad_src = jnp.tile((jnp.arange(npad, dtype=jnp.int32) * 53) % N, (NW, 1))
    pad_dst = jnp.tile(N + (jnp.arange(npad, dtype=jnp.int32) % NPAD),
                       (NW, 1))

    def pad_edges(a):
        return jnp.concatenate(
            [a[0].reshape(NW, EREAL_S), a[1]], axis=1).reshape(NW * PER_S)

    src = pad_edges((edge_index[0], pad_src))
    dst = pad_edges((edge_index[1], pad_dst))
    src2 = jnp.concatenate([src, src + N])
    ea_pad = jnp.concatenate(
        [edge_attr.reshape(NW, EREAL_S, ED),
         jnp.zeros((NW, npad, ED), jnp.float32)], axis=1).reshape(EP, ED)
    zeros = jnp.zeros((N, CW), jnp.float32)

    h0, th0 = _mlp_in(x, W_in, b_in)

    Wcat = jnp.concatenate([We0, We1], axis=1)
    bcat = jnp.concatenate([be0, be1])
    e_all = _edge_feat(ea_pad, Wcat, bcat)


    pre0 = _sc_aggregate(x, th0.reshape(2 * N, CW),
                         e_all[0].reshape(3 * EP, CW), src2, dst, zeros)
    h1, th1 = _mlp_layer0(pre0, W1_0, b1_0, g_0, bt_0, W2_0, b2_0, W3_0, b3_0)

    pre1 = _sc_aggregate(x, th1.reshape(2 * N, CW),
                         e_all[1].reshape(3 * EP, CW), src2, dst, zeros)
    out = _mlp_layer1_out(pre1, h0, h1,
                          W1_1, b1_1, g_1, bt_1, W2_1, b2_1, W3_1, b3_1,
                          W_out, b_out)
    return out


# speedup vs baseline: 3.0365x; 1.3662x over previous
import functools

import jax
import jax.numpy as jnp
from jax import lax
from jax.experimental import pallas as pl
from jax.experimental.pallas import tpu as pltpu
from jax.experimental.pallas import tpu_sc as plsc

N = 10000
E = 160000
DIN = 128
H = 256
ED = 16
DOUT = 128
XC = DIN + H
HID2 = 2 * H
CW = 128

NC = 2
NS = 16
NW = NC * NS
LANES = 16
EB = 64
PER_S = 5184
NCH = PER_S // EB
EREAL_S = E // NW
EP = NW * PER_S
NPAD = 8
ROWS_S = 624
ROWS_LAST = N - (NS - 1) * ROWS_S
OFF_LAST = (NS - 1) * ROWS_S

def _dot(a, b):
    return jax.lax.dot_general(a, b, (((1,), (0,)), ((), ())),
                               preferred_element_type=jnp.float32)


def _mlp_in_body(x_ref, w_ref, b_ref, h_ref, t_ref):
    h = _dot(x_ref[...], w_ref[...]) + b_ref[...]
    h_ref[...] = h
    t_ref[0] = h[:, :CW]
    t_ref[1] = h[:, CW:]


def _mlp_in(x, W_in, b_in, blk=1000):
    grid = (N // blk,)
    return pl.pallas_call(
        _mlp_in_body,
        grid=grid,
        in_specs=[
            pl.BlockSpec((blk, DIN), lambda i: (i, 0)),
            pl.BlockSpec((DIN, H), lambda i: (0, 0)),
            pl.BlockSpec((1, H), lambda i: (0, 0)),
        ],
        out_specs=[
            pl.BlockSpec((blk, H), lambda i: (i, 0)),
            pl.BlockSpec((2, blk, CW), lambda i: (0, i, 0)),
        ],
        out_shape=[
            jax.ShapeDtypeStruct((N, H), jnp.float32),
            jax.ShapeDtypeStruct((2, N, CW), jnp.float32),
        ],
    )(x, W_in, b_in.reshape(1, H))


def _edge_feat_body(ea_ref, w_ref, b_ref, out_ref):
    ea = ea_ref[...]
    for l in range(2):
        for ch in range(3):
            col = l * XC + ch * CW
            out_ref[l, ch] = (_dot(ea, w_ref[:, col:col + CW])
                              + b_ref[:, col:col + CW])


def _edge_feat(edge_attr, Wcat, bcat, blk=2048):
    grid = (EP // blk,)
    return pl.pallas_call(
        _edge_feat_body,
        grid=grid,
        in_specs=[
            pl.BlockSpec((blk, ED), lambda i: (i, 0)),
            pl.BlockSpec((ED, 2 * XC), lambda i: (0, 0)),
            pl.BlockSpec((1, 2 * XC), lambda i: (0, 0)),
        ],
        out_specs=pl.BlockSpec((2, 3, blk, CW), lambda i: (0, 0, i, 0)),
        out_shape=jax.ShapeDtypeStruct((2, 3, EP, CW), jnp.float32),
    )(edge_attr, Wcat, bcat.reshape(1, 2 * XC))


def _ranged_copy(s, mk_src, mk_dst):
    @pl.when(s < NS - 1)
    def _():
        pltpu.sync_copy(mk_src(s * ROWS_S, ROWS_S), mk_dst(s * ROWS_S, ROWS_S))

    @pl.when(s == NS - 1)
    def _():
        pltpu.sync_copy(mk_src(OFF_LAST, ROWS_LAST), mk_dst(OFF_LAST, ROWS_LAST))


def _sc_body(tx_hbm, th_hbm, e_hbm, src2_hbm, dst_hbm, z_hbm, out_hbm,
             srcb, dstb, ebuf, acc, semE, semG, semD, semS, semI):
    c = lax.axis_index("c")
    s = lax.axis_index("s")
    w = c * NS + s

    for ch in range(3):
        tbl = tx_hbm if ch == 0 else th_hbm
        trow0 = 0 if ch < 2 else N

        idx0 = (0 if ch < 2 else EP) + w * PER_S
        dst0 = w * PER_S

        @pl.when(c == 0)
        def _():
            _ranged_copy(s, lambda o, n: tbl.at[pl.ds(trow0 + o, n)],
                         lambda o, n: acc.at[pl.ds(o, n)])

        @pl.when(c == 1)
        def _():
            _ranged_copy(s, lambda o, n: z_hbm.at[pl.ds(o, n)],
                         lambda o, n: acc.at[pl.ds(o, n)])

        plsc.subcore_barrier()

        def e_slice(k):
            return e_hbm.at[pl.ds(ch * EP + dst0 + k * EB, EB)]

        def fetch_ed(k, b):
            pltpu.async_copy(e_slice(k), ebuf.at[b], semE[b])
            pltpu.async_copy(dst_hbm.at[pl.ds(dst0 + k * EB, EB)],
                             dstb.at[b], semD[b])
            pltpu.async_copy(src2_hbm.at[pl.ds(idx0 + k * EB, EB)],
                             srcb.at[b], semI[b])

        def wait_e(k, b):
            pltpu.make_async_copy(e_slice(k), ebuf.at[b], semE[b]).wait()

        def gather_add(k, b):
            pltpu.make_async_copy(src2_hbm.at[pl.ds(idx0 + k * EB, EB)],
                                  srcb.at[b], semI[b]).wait()
            pltpu.async_copy(tbl.at[srcb.at[b]], ebuf.at[b], semG[b],
                             add=True)

        def wait_gather(k, b):
            pltpu.make_async_copy(tbl.at[srcb.at[b]], ebuf.at[b],
                                  semG[b]).wait()

        def scatter(k, b):
            pltpu.make_async_copy(dst_hbm.at[pl.ds(dst0 + k * EB, EB)],
                                  dstb.at[b], semD[b]).wait()
            pltpu.async_copy(ebuf.at[b], acc.at[dstb.at[b]], semS[b],
                             add=True)

        def wait_scatter(k, b):
            pltpu.make_async_copy(ebuf.at[b], acc.at[dstb.at[b]],
                                  semS[b]).wait()

        def relu(b):
            def row(i, carry2):
                for j in range(CW // LANES):
                    sl = pl.ds(j * LANES, LANES)
                    ebuf[b, i, sl] = jnp.maximum(ebuf[b, i, sl], 0.0)
                return carry2

            lax.fori_loop(0, EB, row, 0, unroll=2)

        fetch_ed(0, 0)
        fetch_ed(1, 1)
        wait_e(0, 0)
        gather_add(0, 0)

        def group(g, carry):
            for j in range(3):
                k = 3 * g + j
                b, b1, b2 = j, (j + 1) % 3, (j + 2) % 3

                @pl.when(k <= NCH - 2)
                def _():
                    wait_e(k + 1, b1)
                    gather_add(k + 1, b1)

                wait_gather(k, b)
                relu(b)
                scatter(k, b)

                @pl.when(k >= 1)
                def _():
                    wait_scatter(k - 1, b2)

                @pl.when(k <= NCH - 3)
                def _():
                    fetch_ed(k + 2, b2)
            return carry

        lax.fori_loop(0, NCH // 3, group, 0)
        wait_scatter(NCH - 1, (NCH - 1) % 3)

        plsc.subcore_barrier()

        _ranged_copy(s, lambda o, n: acc.at[pl.ds(o, n)],
                     lambda o, n: out_hbm.at[c, ch, pl.ds(o, n)])
        plsc.subcore_barrier()


@functools.partial(
    pl.kernel,
    out_type=jax.ShapeDtypeStruct((NC, 3, N, CW), jnp.float32),
    mesh=plsc.VectorSubcoreMesh(core_axis_name="c", subcore_axis_name="s",
                                num_cores=NC, num_subcores=NS),
    scratch_types=[
        pltpu.VMEM((3, EB), jnp.int32),
        pltpu.VMEM((3, EB), jnp.int32),
        pltpu.VMEM((3, EB, CW), jnp.float32),
        pltpu.VMEM_SHARED((N + NPAD, CW), jnp.float32),
    ] + [pltpu.SemaphoreType.DMA] * 15,
)
def _sc_aggregate(tx_hbm, th_hbm, e_hbm, src2_hbm, dst_hbm, z_hbm, out_hbm,
                  srcb, dstb, ebuf, acc, *sems):
    _sc_body(tx_hbm, th_hbm, e_hbm, src2_hbm, dst_hbm, z_hbm, out_hbm,
             srcb, dstb, ebuf, acc,
             sems[0:3], sems[3:6], sems[6:9], sems[9:12], sems[12:15])


def _node_mlp(pre_refs, w1_refs, p):
    h = p['b1']
    for ch in range(3):
        pre = pre_refs[ch][...] + pre_refs[3 + ch][...]
        h = h + _dot(pre, w1_refs[ch][...])
    mu = jnp.mean(h, axis=-1, keepdims=True)
    var = jnp.mean((h - mu) ** 2, axis=-1, keepdims=True)
    h = p['g'] * (h - mu) / jnp.sqrt(var + 1e-5) + p['bt']
    h = jnp.where(h >= 0, h, 0.1 * h)
    h = _dot(h, p['W2']) + p['b2']
    h = jnp.where(h >= 0, h, 0.1 * h)
    return _dot(h, p['W3']) + p['b3']


def _mlp0_body(p00, p01, p02, p10, p11, p12,
               w1a, w1b, w1c, b1, g, bt, w2, b2, w3, b3,
               h_ref, t_ref):
    p = dict(b1=b1[...], g=g[...], bt=bt[...], W2=w2[...], b2=b2[...],
             W3=w3[...], b3=b3[...])
    h = _node_mlp((p00, p01, p02, p10, p11, p12), (w1a, w1b, w1c), p)
    h_ref[...] = h
    t_ref[0] = h[:, :CW]
    t_ref[1] = h[:, CW:]


def _pre_specs(blk):
    return [pl.BlockSpec((blk, CW), lambda i: (i, 0)) for _ in range(6)]


def _wspec(shp):
    return pl.BlockSpec(shp, lambda i: (0, 0))


def _mlp_layer0(pre, W1, b1, g, bt, W2, b2, W3, b3, blk=1000):
    grid = (N // blk,)
    return pl.pallas_call(
        _mlp0_body,
        grid=grid,
        in_specs=_pre_specs(blk) + [
            _wspec((CW, HID2)), _wspec((CW, HID2)), _wspec((CW, HID2)),
            _wspec((1, HID2)), _wspec((1, HID2)), _wspec((1, HID2)),
            _wspec((HID2, HID2)), _wspec((1, HID2)),
            _wspec((HID2, H)), _wspec((1, H)),
        ],
        out_specs=[
            pl.BlockSpec((blk, H), lambda i: (i, 0)),
            pl.BlockSpec((2, blk, CW), lambda i: (0, i, 0)),
        ],
        out_shape=[
            jax.ShapeDtypeStruct((N, H), jnp.float32),
            jax.ShapeDtypeStruct((2, N, CW), jnp.float32),
        ],
    )(pre[0, 0], pre[0, 1], pre[0, 2], pre[1, 0], pre[1, 1], pre[1, 2],
      W1[:CW], W1[CW:2 * CW], W1[2 * CW:], b1.reshape(1, HID2),
      g.reshape(1, HID2), bt.reshape(1, HID2), W2, b2.reshape(1, HID2),
      W3, b3.reshape(1, H))


def _mlp1_body(p00, p01, p02, p10, p11, p12, h0_ref, h1_ref,
               w1a, w1b, w1c, b1, g, bt, w2, b2, w3, b3,
               wo0, wo1, wo2, bo, out_ref):
    p = dict(b1=b1[...], g=g[...], bt=bt[...], W2=w2[...], b2=b2[...],
             W3=w3[...], b3=b3[...])
    h2 = _node_mlp((p00, p01, p02, p10, p11, p12), (w1a, w1b, w1c), p)
    out_ref[...] = (_dot(h0_ref[...], wo0[...]) + _dot(h1_ref[...], wo1[...])
                    + _dot(h2, wo2[...]) + bo[...])


def _mlp_layer1_out(pre, h0, h1, W1, b1, g, bt, W2, b2, W3, b3,
                    W_out, b_out, blk=1000):
    grid = (N // blk,)
    return pl.pallas_call(
        _mlp1_body,
        grid=grid,
        in_specs=_pre_specs(blk) + [
            pl.BlockSpec((blk, H), lambda i: (i, 0)),
            pl.BlockSpec((blk, H), lambda i: (i, 0)),
            _wspec((CW, HID2)), _wspec((CW, HID2)), _wspec((CW, HID2)),
            _wspec((1, HID2)), _wspec((1, HID2)), _wspec((1, HID2)),
            _wspec((HID2, HID2)), _wspec((1, HID2)),
            _wspec((HID2, H)), _wspec((1, H)),
            _wspec((H, DOUT)), _wspec((H, DOUT)), _wspec((H, DOUT)),
            _wspec((1, DOUT)),
        ],
        out_specs=pl.BlockSpec((blk, DOUT), lambda i: (i, 0)),
        out_shape=jax.ShapeDtypeStruct((N, DOUT), jnp.float32),
    )(pre[0, 0], pre[0, 1], pre[0, 2], pre[1, 0], pre[1, 1], pre[1, 2],
      h0, h1,
      W1[:CW], W1[CW:2 * CW], W1[2 * CW:], b1.reshape(1, HID2),
      g.reshape(1, HID2), bt.reshape(1, HID2), W2, b2.reshape(1, HID2),
      W3, b3.reshape(1, H),
      W_out[:H], W_out[H:2 * H], W_out[2 * H:], b_out.reshape(1, DOUT))


def kernel(x, edge_index, edge_attr, W_in, b_in,
           We0, be0, W1_0, b1_0, g_0, bt_0, W2_0, b2_0, W3_0, b3_0,
           We1, be1, W1_1, b1_1, g_1, bt_1, W2_1, b2_1, W3_1, b3_1,
           W_out, b_out):
    npad = PER_S - EREAL_S
    pad_src = jnp.tile((jnp.arange(npad, dtype=jnp.int32) * 53) % N, (NW, 1))
    pad_dst = jnp.tile(N + (jnp.arange(npad, dtype=jnp.int32) % NPAD),
                       (NW, 1))

    def pad_edges(a):
        return jnp.concatenate(
            [a[0].reshape(NW, EREAL_S), a[1]], axis=1).reshape(NW * PER_S)

    src = pad_edges((edge_index[0], pad_src))
    dst = pad_edges((edge_index[1], pad_dst))
    src2 = jnp.concatenate([src, src + N])
    ea_pad = jnp.concatenate(
        [edge_attr.reshape(NW, EREAL_S, ED),
         jnp.zeros((NW, npad, ED), jnp.float32)], axis=1).reshape(EP, ED)
    zeros = jnp.zeros((N, CW), jnp.float32)

    h0, th0 = _mlp_in(x, W_in, b_in)

    Wcat = jnp.concatenate([We0, We1], axis=1)
    bcat = jnp.concatenate([be0, be1])
    e_all = _edge_feat(ea_pad, Wcat, bcat)


    pre0 = _sc_aggregate(x, th0.reshape(2 * N, CW),
                         e_all[0].reshape(3 * EP, CW), src2, dst, zeros)
    h1, th1 = _mlp_layer0(pre0, W1_0, b1_0, g_0, bt_0, W2_0, b2_0, W3_0, b3_0)

    pre1 = _sc_aggregate(x, th1.reshape(2 * N, CW),
                         e_all[1].reshape(3 * EP, CW), src2, dst, zeros)
    out = _mlp_layer1_out(pre1, h0, h1,
                          W1_1, b1_1, g_1, bt_1, W2_1, b2_1, W3_1, b3_1,
                          W_out, b_out)
    return out


# speedup vs baseline: 3.9078x; 1.2869x over previous
import functools

import jax
import jax.numpy as jnp
from jax import lax
from jax.experimental import pallas as pl
from jax.experimental.pallas import tpu as pltpu
from jax.experimental.pallas import tpu_sc as plsc

N = 10000
E = 160000
DIN = 128
H = 256
ED = 16
DOUT = 128
XC = DIN + H
HID2 = 2 * H
CW = 128

NC = 2
NS = 16
NW = NC * NS
LANES = 16
EB = 64
PER_S = E // NW
NCH = 79
NRING = 78
TAILR = PER_S - NRING * EB
EPAD = 162000
ROWS_S = 624
ROWS_LAST = N - (NS - 1) * ROWS_S
OFF_LAST = (NS - 1) * ROWS_S

def _dot(a, b):
    return jax.lax.dot_general(a, b, (((1,), (0,)), ((), ())),
                               preferred_element_type=jnp.float32)


def _mlp_in_body(x_ref, w_ref, b_ref, h_ref, t_ref):
    h = _dot(x_ref[...], w_ref[...]) + b_ref[...]
    h_ref[...] = h
    t_ref[0] = h[:, :CW]
    t_ref[1] = h[:, CW:]


def _mlp_in(x, W_in, b_in, blk=1000):
    grid = (N // blk,)
    return pl.pallas_call(
        _mlp_in_body,
        grid=grid,
        in_specs=[
            pl.BlockSpec((blk, DIN), lambda i: (i, 0)),
            pl.BlockSpec((DIN, H), lambda i: (0, 0)),
            pl.BlockSpec((1, H), lambda i: (0, 0)),
        ],
        out_specs=[
            pl.BlockSpec((blk, H), lambda i: (i, 0)),
            pl.BlockSpec((2, blk, CW), lambda i: (0, i, 0)),
        ],
        out_shape=[
            jax.ShapeDtypeStruct((N, H), jnp.float32),
            jax.ShapeDtypeStruct((2, N, CW), jnp.float32),
        ],
    )(x, W_in, b_in.reshape(1, H))


def _edge_feat_body(ea_ref, w_ref, b_ref, out_ref):
    ea = ea_ref[...]
    for ch in range(3):
        col = ch * CW
        out_ref[ch] = (_dot(ea, w_ref[:, col:col + CW])
                       + b_ref[:, col:col + CW])


def _edge_feat(ea_pad, We, be, blk=2000):
    grid = (EPAD // blk,)
    return pl.pallas_call(
        _edge_feat_body,
        grid=grid,
        in_specs=[
            pl.BlockSpec((blk, ED), lambda i: (i, 0)),
            pl.BlockSpec((ED, XC), lambda i: (0, 0)),
            pl.BlockSpec((1, XC), lambda i: (0, 0)),
        ],
        out_specs=pl.BlockSpec((3, blk, CW), lambda i: (0, i, 0)),
        out_shape=jax.ShapeDtypeStruct((3, EPAD, CW), jnp.float32),
    )(ea_pad, We, be.reshape(1, XC))


def _ranged_copy(s, mk_src, mk_dst):
    @pl.when(s < NS - 1)
    def _():
        pltpu.sync_copy(mk_src(s * ROWS_S, ROWS_S), mk_dst(s * ROWS_S, ROWS_S))

    @pl.when(s == NS - 1)
    def _():
        pltpu.sync_copy(mk_src(OFF_LAST, ROWS_LAST), mk_dst(OFF_LAST, ROWS_LAST))


def _sc_body(tx_hbm, th_hbm, e_hbm, src2_hbm, dst_hbm, z_hbm, out_hbm,
             srcb, dstb, ebuf, acc, semE, semG, semD, semS, semI):
    c = lax.axis_index("c")
    s = lax.axis_index("s")
    w = c * NS + s

    for ch in range(3):
        tbl = tx_hbm if ch == 0 else th_hbm
        trow0 = 0 if ch < 2 else N

        idx0 = (0 if ch < 2 else EPAD) + w * PER_S
        dst0 = w * PER_S

        @pl.when(c == 0)
        def _():
            _ranged_copy(s, lambda o, n: tbl.at[pl.ds(trow0 + o, n)],
                         lambda o, n: acc.at[pl.ds(o, n)])

        @pl.when(c == 1)
        def _():
            _ranged_copy(s, lambda o, n: z_hbm.at[pl.ds(o, n)],
                         lambda o, n: acc.at[pl.ds(o, n)])

        plsc.subcore_barrier()

        def e_slice(k):
            return e_hbm.at[pl.ds(ch * EPAD + dst0 + k * EB, EB)]

        def fetch_ed(k, b):
            pltpu.async_copy(e_slice(k), ebuf.at[b], semE[b])
            pltpu.async_copy(dst_hbm.at[pl.ds(dst0 + k * EB, EB)],
                             dstb.at[b], semD[b])
            pltpu.async_copy(src2_hbm.at[pl.ds(idx0 + k * EB, EB)],
                             srcb.at[b], semI[b])

        def wait_e(k, b):
            pltpu.make_async_copy(e_slice(k), ebuf.at[b], semE[b]).wait()

        def gather_add(k, b):
            pltpu.make_async_copy(src2_hbm.at[pl.ds(idx0 + k * EB, EB)],
                                  srcb.at[b], semI[b]).wait()
            pltpu.async_copy(tbl.at[srcb.at[b]], ebuf.at[b], semG[b],
                             add=True)

        def wait_gather(k, b):
            pltpu.make_async_copy(tbl.at[srcb.at[b]], ebuf.at[b],
                                  semG[b]).wait()

        def scatter(k, b):
            pltpu.make_async_copy(dst_hbm.at[pl.ds(dst0 + k * EB, EB)],
                                  dstb.at[b], semD[b]).wait()
            pltpu.async_copy(ebuf.at[b], acc.at[dstb.at[b]], semS[b],
                             add=True)

        def wait_scatter(k, b):
            pltpu.make_async_copy(ebuf.at[b], acc.at[dstb.at[b]],
                                  semS[b]).wait()

        def relu(b):
            def row(i, carry2):
                for j in range(CW // LANES):
                    sl = pl.ds(j * LANES, LANES)
                    ebuf[b, i, sl] = jnp.maximum(ebuf[b, i, sl], 0.0)
                return carry2

            lax.fori_loop(0, EB, row, 0, unroll=2)

        fetch_ed(0, 0)
        fetch_ed(1, 1)
        wait_e(0, 0)
        gather_add(0, 0)

        def group(g, carry):
            for j in range(3):
                k = 3 * g + j
                b, b1, b2 = j, (j + 1) % 3, (j + 2) % 3

                @pl.when(k <= NRING - 2)
                def _():
                    wait_e(k + 1, b1)
                    gather_add(k + 1, b1)

                wait_gather(k, b)
                relu(b)
                scatter(k, b)

                @pl.when(k >= 1)
                def _():
                    wait_scatter(k - 1, b2)

                @pl.when(k <= NRING - 3)
                def _():
                    fetch_ed(k + 2, b2)
            return carry

        lax.fori_loop(0, NRING // 3, group, 0)
        wait_scatter(NRING - 1, (NRING - 1) % 3)

        kt = NRING
        fetch_ed(kt, 0)
        wait_e(kt, 0)
        gather_add(kt, 0)
        wait_gather(kt, 0)

        def tail_row(i, carry2):
            for j in range(CW // LANES):
                sl = pl.ds(j * LANES, LANES)
                ebuf[0, i, sl] = jnp.maximum(ebuf[0, i, sl], 0.0)
            return carry2

        lax.fori_loop(0, TAILR, tail_row, 0)

        def zero_row(i, carry2):
            for j in range(CW // LANES):
                ebuf[0, i, pl.ds(j * LANES, LANES)] = jnp.zeros(
                    (LANES,), jnp.float32)
            return carry2

        lax.fori_loop(TAILR, EB, zero_row, 0)
        scatter(kt, 0)
        wait_scatter(kt, 0)

        plsc.subcore_barrier()

        _ranged_copy(s, lambda o, n: acc.at[pl.ds(o, n)],
                     lambda o, n: out_hbm.at[c, ch, pl.ds(o, n)])
        plsc.subcore_barrier()


@functools.partial(
    pl.kernel,
    out_type=jax.ShapeDtypeStruct((NC, 3, N, CW), jnp.float32),
    mesh=plsc.VectorSubcoreMesh(core_axis_name="c", subcore_axis_name="s",
                                num_cores=NC, num_subcores=NS),
    scratch_types=[
        pltpu.VMEM((3, EB), jnp.int32),
        pltpu.VMEM((3, EB), jnp.int32),
        pltpu.VMEM((3, EB, CW), jnp.float32),
        pltpu.VMEM_SHARED((N, CW), jnp.float32),
    ] + [pltpu.SemaphoreType.DMA] * 15,
)
def _sc_aggregate(tx_hbm, th_hbm, e_hbm, src2_hbm, dst_hbm, z_hbm, out_hbm,
                  srcb, dstb, ebuf, acc, *sems):
    _sc_body(tx_hbm, th_hbm, e_hbm, src2_hbm, dst_hbm, z_hbm, out_hbm,
             srcb, dstb, ebuf, acc,
             sems[0:3], sems[3:6], sems[6:9], sems[9:12], sems[12:15])


def _node_mlp(pre_refs, w1_refs, p):
    h = p['b1']
    for ch in range(3):
        pre = pre_refs[ch][...] + pre_refs[3 + ch][...]
        h = h + _dot(pre, w1_refs[ch][...])
    mu = jnp.mean(h, axis=-1, keepdims=True)
    var = jnp.mean((h - mu) ** 2, axis=-1, keepdims=True)
    h = p['g'] * (h - mu) / jnp.sqrt(var + 1e-5) + p['bt']
    h = jnp.where(h >= 0, h, 0.1 * h)
    h = _dot(h, p['W2']) + p['b2']
    h = jnp.where(h >= 0, h, 0.1 * h)
    return _dot(h, p['W3']) + p['b3']


def _mlp0_body(p00, p01, p02, p10, p11, p12,
               w1a, w1b, w1c, b1, g, bt, w2, b2, w3, b3,
               h_ref, t_ref):
    p = dict(b1=b1[...], g=g[...], bt=bt[...], W2=w2[...], b2=b2[...],
             W3=w3[...], b3=b3[...])
    h = _node_mlp((p00, p01, p02, p10, p11, p12), (w1a, w1b, w1c), p)
    h_ref[...] = h
    t_ref[0] = h[:, :CW]
    t_ref[1] = h[:, CW:]


def _pre_specs(blk):
    return [pl.BlockSpec((blk, CW), lambda i: (i, 0)) for _ in range(6)]


def _wspec(shp):
    return pl.BlockSpec(shp, lambda i: (0, 0))


def _mlp_layer0(pre, W1, b1, g, bt, W2, b2, W3, b3, blk=1000):
    grid = (N // blk,)
    return pl.pallas_call(
        _mlp0_body,
        grid=grid,
        in_specs=_pre_specs(blk) + [
            _wspec((CW, HID2)), _wspec((CW, HID2)), _wspec((CW, HID2)),
            _wspec((1, HID2)), _wspec((1, HID2)), _wspec((1, HID2)),
            _wspec((HID2, HID2)), _wspec((1, HID2)),
            _wspec((HID2, H)), _wspec((1, H)),
        ],
        out_specs=[
            pl.BlockSpec((blk, H), lambda i: (i, 0)),
            pl.BlockSpec((2, blk, CW), lambda i: (0, i, 0)),
        ],
        out_shape=[
            jax.ShapeDtypeStruct((N, H), jnp.float32),
            jax.ShapeDtypeStruct((2, N, CW), jnp.float32),
        ],
    )(pre[0, 0], pre[0, 1], pre[0, 2], pre[1, 0], pre[1, 1], pre[1, 2],
      W1[:CW], W1[CW:2 * CW], W1[2 * CW:], b1.reshape(1, HID2),
      g.reshape(1, HID2), bt.reshape(1, HID2), W2, b2.reshape(1, HID2),
      W3, b3.reshape(1, H))


def _mlp1_body(p00, p01, p02, p10, p11, p12, h0_ref, h1_ref,
               w1a, w1b, w1c, b1, g, bt, w2, b2, w3, b3,
               wo0, wo1, wo2, bo, out_ref):
    p = dict(b1=b1[...], g=g[...], bt=bt[...], W2=w2[...], b2=b2[...],
             W3=w3[...], b3=b3[...])
    h2 = _node_mlp((p00, p01, p02, p10, p11, p12), (w1a, w1b, w1c), p)
    out_ref[...] = (_dot(h0_ref[...], wo0[...]) + _dot(h1_ref[...], wo1[...])
                    + _dot(h2, wo2[...]) + bo[...])


def _mlp_layer1_out(pre, h0, h1, W1, b1, g, bt, W2, b2, W3, b3,
                    W_out, b_out, blk=1000):
    grid = (N // blk,)
    return pl.pallas_call(
        _mlp1_body,
        grid=grid,
        in_specs=_pre_specs(blk) + [
            pl.BlockSpec((blk, H), lambda i: (i, 0)),
            pl.BlockSpec((blk, H), lambda i: (i, 0)),
            _wspec((CW, HID2)), _wspec((CW, HID2)), _wspec((CW, HID2)),
            _wspec((1, HID2)), _wspec((1, HID2)), _wspec((1, HID2)),
            _wspec((HID2, HID2)), _wspec((1, HID2)),
            _wspec((HID2, H)), _wspec((1, H)),
            _wspec((H, DOUT)), _wspec((H, DOUT)), _wspec((H, DOUT)),
            _wspec((1, DOUT)),
        ],
        out_specs=pl.BlockSpec((blk, DOUT), lambda i: (i, 0)),
        out_shape=jax.ShapeDtypeStruct((N, DOUT), jnp.float32),
    )(pre[0, 0], pre[0, 1], pre[0, 2], pre[1, 0], pre[1, 1], pre[1, 2],
      h0, h1,
      W1[:CW], W1[CW:2 * CW], W1[2 * CW:], b1.reshape(1, HID2),
      g.reshape(1, HID2), bt.reshape(1, HID2), W2, b2.reshape(1, HID2),
      W3, b3.reshape(1, H),
      W_out[:H], W_out[H:2 * H], W_out[2 * H:], b_out.reshape(1, DOUT))


def kernel(x, edge_index, edge_attr, W_in, b_in,
           We0, be0, W1_0, b1_0, g_0, bt_0, W2_0, b2_0, W3_0, b3_0,
           We1, be1, W1_1, b1_1, g_1, bt_1, W2_1, b2_1, W3_1, b3_1,
           W_out, b_out):
    npad = EPAD - E
    src = jnp.concatenate([edge_index[0], jnp.zeros((npad,), jnp.int32)])
    dst = jnp.concatenate([edge_index[1], jnp.zeros((npad,), jnp.int32)])
    src2 = jnp.concatenate([src, src + N])
    ea_pad = jnp.concatenate([edge_attr, jnp.zeros((npad, ED), jnp.float32)])
    zeros = jnp.zeros((N, CW), jnp.float32)

    h0, th0 = _mlp_in(x, W_in, b_in)

    e0 = _edge_feat(ea_pad, We0, be0)
    e1 = _edge_feat(ea_pad, We1, be1)

    pre0 = _sc_aggregate(x, th0.reshape(2 * N, CW),
                         e0.reshape(3 * EPAD, CW), src2, dst, zeros)
    h1, th1 = _mlp_layer0(pre0, W1_0, b1_0, g_0, bt_0, W2_0, b2_0, W3_0, b3_0)

    pre1 = _sc_aggregate(x, th1.reshape(2 * N, CW),
                         e1.reshape(3 * EPAD, CW), src2, dst, zeros)
    out = _mlp_layer1_out(pre1, h0, h1,
                          W1_1, b1_1, g_1, bt_1, W2_1, b2_1, W3_1, b3_1,
                          W_out, b_out)
    return out


# speedup vs baseline: 4.0949x; 1.0479x over previous
import functools

import jax
import jax.numpy as jnp
from jax import lax
from jax.experimental import pallas as pl
from jax.experimental.pallas import tpu as pltpu
from jax.experimental.pallas import tpu_sc as plsc

N = 10000
E = 160000
DIN = 128
H = 256
ED = 16
DOUT = 128
XC = DIN + H
HID2 = 2 * H
CW = 128

NC = 2
NS = 16
NW = NC * NS
LANES = 16
EB = 64
PER_S = E // NW
NCH = 79
NRING = 78
TAILR = PER_S - NRING * EB
EPAD = 162000
ROWS_S = 624
ROWS_LAST = N - (NS - 1) * ROWS_S
OFF_LAST = (NS - 1) * ROWS_S

def _dot(a, b):
    return jax.lax.dot_general(a, b, (((1,), (0,)), ((), ())),
                               preferred_element_type=jnp.float32)


def _mlp_in_body(x_ref, w_ref, b_ref, h_ref, t_ref):
    h = _dot(x_ref[...], w_ref[...]) + b_ref[...]
    h_ref[...] = h
    t_ref[0] = h[:, :CW]
    t_ref[1] = h[:, CW:]


def _mlp_in(x, W_in, b_in, blk=1000):
    grid = (N // blk,)
    return pl.pallas_call(
        _mlp_in_body,
        grid=grid,
        in_specs=[
            pl.BlockSpec((blk, DIN), lambda i: (i, 0)),
            pl.BlockSpec((DIN, H), lambda i: (0, 0)),
            pl.BlockSpec((1, H), lambda i: (0, 0)),
        ],
        out_specs=[
            pl.BlockSpec((blk, H), lambda i: (i, 0)),
            pl.BlockSpec((2, blk, CW), lambda i: (0, i, 0)),
        ],
        out_shape=[
            jax.ShapeDtypeStruct((N, H), jnp.float32),
            jax.ShapeDtypeStruct((2, N, CW), jnp.float32),
        ],
    )(x, W_in, b_in.reshape(1, H))


def _edge_feat_body(nch, ea_ref, w_ref, b_ref, out_ref):
    ea = ea_ref[...]
    for ch in range(nch):
        col = ch * CW
        out_ref[ch] = (_dot(ea, w_ref[:, col:col + CW])
                       + b_ref[:, col:col + CW])


def _edge_feat(ea, We, be, blk=2000):
    nch = We.shape[1] // CW
    grid = (EPAD // blk,)
    last = E // blk - 1
    return pl.pallas_call(
        functools.partial(_edge_feat_body, nch),
        grid=grid,
        in_specs=[
            pl.BlockSpec((blk, ED), lambda i: (jnp.minimum(i, last), 0)),
            pl.BlockSpec((ED, nch * CW), lambda i: (0, 0)),
            pl.BlockSpec((1, nch * CW), lambda i: (0, 0)),
        ],
        out_specs=pl.BlockSpec((nch, blk, CW), lambda i: (0, i, 0)),
        out_shape=jax.ShapeDtypeStruct((nch, EPAD, CW), jnp.float32),
    )(ea, We, be.reshape(1, nch * CW))


def _ranged_copy(s, mk_src, mk_dst):
    @pl.when(s < NS - 1)
    def _():
        pltpu.sync_copy(mk_src(s * ROWS_S, ROWS_S), mk_dst(s * ROWS_S, ROWS_S))

    @pl.when(s == NS - 1)
    def _():
        pltpu.sync_copy(mk_src(OFF_LAST, ROWS_LAST), mk_dst(OFF_LAST, ROWS_LAST))


def _sc_body(nph, tbl, e_hbm, idx_hbm, dst_hbm, z_hbm, out_hbm,
             srcb, dstb, ebuf, acc, semE, semG, semD, semS, semI):
    c = lax.axis_index("c")
    s = lax.axis_index("s")
    w = c * NS + s

    for ch in range(nph):
        trow0 = ch * N
        idx0 = ch * EPAD + w * PER_S
        dst0 = w * PER_S

        @pl.when(c == 0)
        def _():
            _ranged_copy(s, lambda o, n: tbl.at[pl.ds(trow0 + o, n)],
                         lambda o, n: acc.at[pl.ds(o, n)])

        @pl.when(c == 1)
        def _():
            _ranged_copy(s, lambda o, n: z_hbm.at[pl.ds(o, n)],
                         lambda o, n: acc.at[pl.ds(o, n)])

        plsc.subcore_barrier()

        def e_slice(k):
            return e_hbm.at[pl.ds(ch * EPAD + dst0 + k * EB, EB)]

        def fetch_ed(k, b):
            pltpu.async_copy(e_slice(k), ebuf.at[b], semE[b])
            pltpu.async_copy(dst_hbm.at[pl.ds(dst0 + k * EB, EB)],
                             dstb.at[b], semD[b])
            pltpu.async_copy(idx_hbm.at[pl.ds(idx0 + k * EB, EB)],
                             srcb.at[b], semI[b])

        def wait_e(k, b):
            pltpu.make_async_copy(e_slice(k), ebuf.at[b], semE[b]).wait()

        def gather_add(k, b):
            pltpu.make_async_copy(idx_hbm.at[pl.ds(idx0 + k * EB, EB)],
                                  srcb.at[b], semI[b]).wait()
            pltpu.async_copy(tbl.at[srcb.at[b]], ebuf.at[b], semG[b],
                             add=True)

        def wait_gather(k, b):
            pltpu.make_async_copy(tbl.at[srcb.at[b]], ebuf.at[b],
                                  semG[b]).wait()

        def scatter(k, b):
            pltpu.make_async_copy(dst_hbm.at[pl.ds(dst0 + k * EB, EB)],
                                  dstb.at[b], semD[b]).wait()
            pltpu.async_copy(ebuf.at[b], acc.at[dstb.at[b]], semS[b],
                             add=True)

        def wait_scatter(k, b):
            pltpu.make_async_copy(ebuf.at[b], acc.at[dstb.at[b]],
                                  semS[b]).wait()

        def relu(b):
            def row(i, carry2):
                for j in range(CW // LANES):
                    sl = pl.ds(j * LANES, LANES)
                    ebuf[b, i, sl] = jnp.maximum(ebuf[b, i, sl], 0.0)
                return carry2

            lax.fori_loop(0, EB, row, 0, unroll=2)

        fetch_ed(0, 0)
        fetch_ed(1, 1)
        wait_e(0, 0)
        gather_add(0, 0)

        def group(g, carry):
            for j in range(3):
                k = 3 * g + j
                b, b1, b2 = j, (j + 1) % 3, (j + 2) % 3

                @pl.when(k <= NRING - 2)
                def _():
                    wait_e(k + 1, b1)
                    gather_add(k + 1, b1)

                wait_gather(k, b)
                relu(b)
                scatter(k, b)

                @pl.when(k >= 1)
                def _():
                    wait_scatter(k - 1, b2)

                @pl.when(k <= NRING - 3)
                def _():
                    fetch_ed(k + 2, b2)
            return carry

        lax.fori_loop(0, NRING // 3, group, 0)
        wait_scatter(NRING - 1, (NRING - 1) % 3)

        kt = NRING
        fetch_ed(kt, 0)
        wait_e(kt, 0)
        gather_add(kt, 0)
        wait_gather(kt, 0)

        def tail_row(i, carry2):
            for j in range(CW // LANES):
                sl = pl.ds(j * LANES, LANES)
                ebuf[0, i, sl] = jnp.maximum(ebuf[0, i, sl], 0.0)
            return carry2

        lax.fori_loop(0, TAILR, tail_row, 0)

        def zero_row(i, carry2):
            for j in range(CW // LANES):
                ebuf[0, i, pl.ds(j * LANES, LANES)] = jnp.zeros(
                    (LANES,), jnp.float32)
            return carry2

        lax.fori_loop(TAILR, EB, zero_row, 0)
        scatter(kt, 0)
        wait_scatter(kt, 0)

        plsc.subcore_barrier()

        _ranged_copy(s, lambda o, n: acc.at[pl.ds(o, n)],
                     lambda o, n: out_hbm.at[c, ch, pl.ds(o, n)])
        plsc.subcore_barrier()


def _make_sc_agg(nph):
    @functools.partial(
        pl.kernel,
        out_type=jax.ShapeDtypeStruct((NC, nph, N, CW), jnp.float32),
        mesh=plsc.VectorSubcoreMesh(core_axis_name="c", subcore_axis_name="s",
                                    num_cores=NC, num_subcores=NS),
        scratch_types=[
            pltpu.VMEM((3, EB), jnp.int32),
            pltpu.VMEM((3, EB), jnp.int32),
            pltpu.VMEM((3, EB, CW), jnp.float32),
            pltpu.VMEM_SHARED((N, CW), jnp.float32),
        ] + [pltpu.SemaphoreType.DMA] * 15,
    )
    def agg(tbl_hbm, e_hbm, idx_hbm, dst_hbm, z_hbm, out_hbm,
            srcb, dstb, ebuf, acc, *sems):
        _sc_body(nph, tbl_hbm, e_hbm, idx_hbm, dst_hbm, z_hbm, out_hbm,
                 srcb, dstb, ebuf, acc,
                 sems[0:3], sems[3:6], sems[6:9], sems[9:12], sems[12:15])

    return agg


_sc_agg_x = _make_sc_agg(1)
_sc_agg_h = _make_sc_agg(2)


def _node_mlp(pre_refs, w1_refs, p):
    h = p['b1']
    for ch in range(3):
        pre = pre_refs[ch][...] + pre_refs[3 + ch][...]
        h = h + _dot(pre, w1_refs[ch][...])
    mu = jnp.mean(h, axis=-1, keepdims=True)
    var = jnp.mean((h - mu) ** 2, axis=-1, keepdims=True)
    h = p['g'] * (h - mu) / jnp.sqrt(var + 1e-5) + p['bt']
    h = jnp.where(h >= 0, h, 0.1 * h)
    h = _dot(h, p['W2']) + p['b2']
    h = jnp.where(h >= 0, h, 0.1 * h)
    return _dot(h, p['W3']) + p['b3']


def _mlp0_body(p00, p01, p02, p10, p11, p12,
               w1a, w1b, w1c, b1, g, bt, w2, b2, w3, b3,
               h_ref, t_ref):
    p = dict(b1=b1[...], g=g[...], bt=bt[...], W2=w2[...], b2=b2[...],
             W3=w3[...], b3=b3[...])
    h = _node_mlp((p00, p01, p02, p10, p11, p12), (w1a, w1b, w1c), p)
    h_ref[...] = h
    t_ref[0] = h[:, :CW]
    t_ref[1] = h[:, CW:]


def _pre_specs(blk):
    return [pl.BlockSpec((blk, CW), lambda i: (i, 0)) for _ in range(6)]


def _wspec(shp):
    return pl.BlockSpec(shp, lambda i: (0, 0))


def _mlp_layer0(pres, W1, b1, g, bt, W2, b2, W3, b3, blk=1000):
    grid = (N // blk,)
    return pl.pallas_call(
        _mlp0_body,
        grid=grid,
        in_specs=_pre_specs(blk) + [
            _wspec((CW, HID2)), _wspec((CW, HID2)), _wspec((CW, HID2)),
            _wspec((1, HID2)), _wspec((1, HID2)), _wspec((1, HID2)),
            _wspec((HID2, HID2)), _wspec((1, HID2)),
            _wspec((HID2, H)), _wspec((1, H)),
        ],
        out_specs=[
            pl.BlockSpec((blk, H), lambda i: (i, 0)),
            pl.BlockSpec((2, blk, CW), lambda i: (0, i, 0)),
        ],
        out_shape=[
            jax.ShapeDtypeStruct((N, H), jnp.float32),
            jax.ShapeDtypeStruct((2, N, CW), jnp.float32),
        ],
    )(*pres,
      W1[:CW], W1[CW:2 * CW], W1[2 * CW:], b1.reshape(1, HID2),
      g.reshape(1, HID2), bt.reshape(1, HID2), W2, b2.reshape(1, HID2),
      W3, b3.reshape(1, H))


def _mlp1_body(p00, p01, p02, p10, p11, p12, h0_ref, h1_ref,
               w1a, w1b, w1c, b1, g, bt, w2, b2, w3, b3,
               wo0, wo1, wo2, bo, out_ref):
    p = dict(b1=b1[...], g=g[...], bt=bt[...], W2=w2[...], b2=b2[...],
             W3=w3[...], b3=b3[...])
    h2 = _node_mlp((p00, p01, p02, p10, p11, p12), (w1a, w1b, w1c), p)
    out_ref[...] = (_dot(h0_ref[...], wo0[...]) + _dot(h1_ref[...], wo1[...])
                    + _dot(h2, wo2[...]) + bo[...])


def _mlp_layer1_out(pres, h0, h1, W1, b1, g, bt, W2, b2, W3, b3,
                    W_out, b_out, blk=1000):
    grid = (N // blk,)
    return pl.pallas_call(
        _mlp1_body,
        grid=grid,
        in_specs=_pre_specs(blk) + [
            pl.BlockSpec((blk, H), lambda i: (i, 0)),
            pl.BlockSpec((blk, H), lambda i: (i, 0)),
            _wspec((CW, HID2)), _wspec((CW, HID2)), _wspec((CW, HID2)),
            _wspec((1, HID2)), _wspec((1, HID2)), _wspec((1, HID2)),
            _wspec((HID2, HID2)), _wspec((1, HID2)),
            _wspec((HID2, H)), _wspec((1, H)),
            _wspec((H, DOUT)), _wspec((H, DOUT)), _wspec((H, DOUT)),
            _wspec((1, DOUT)),
        ],
        out_specs=pl.BlockSpec((blk, DOUT), lambda i: (i, 0)),
        out_shape=jax.ShapeDtypeStruct((N, DOUT), jnp.float32),
    )(*pres,
      h0, h1,
      W1[:CW], W1[CW:2 * CW], W1[2 * CW:], b1.reshape(1, HID2),
      g.reshape(1, HID2), bt.reshape(1, HID2), W2, b2.reshape(1, HID2),
      W3, b3.reshape(1, H),
      W_out[:H], W_out[H:2 * H], W_out[2 * H:], b_out.reshape(1, DOUT))


def kernel(x, edge_index, edge_attr, W_in, b_in,
           We0, be0, W1_0, b1_0, g_0, bt_0, W2_0, b2_0, W3_0, b3_0,
           We1, be1, W1_1, b1_1, g_1, bt_1, W2_1, b2_1, W3_1, b3_1,
           W_out, b_out):
    npad = EPAD - E
    src = jnp.concatenate([edge_index[0], jnp.zeros((npad,), jnp.int32)])
    dst = jnp.concatenate([edge_index[1], jnp.zeros((npad,), jnp.int32)])
    src2 = jnp.concatenate([src, src + N])
    zeros = jnp.zeros((N, CW), jnp.float32)

    h0, th0 = _mlp_in(x, W_in, b_in)

    e0x = _edge_feat(edge_attr, We0[:, :CW], be0[:CW])
    e0h = _edge_feat(edge_attr, We0[:, CW:], be0[CW:])
    e1x = _edge_feat(edge_attr, We1[:, :CW], be1[:CW])
    e1h = _edge_feat(edge_attr, We1[:, CW:], be1[CW:])

    prex0 = _sc_agg_x(x, e0x.reshape(EPAD, CW), src, dst, zeros)
    preh0 = _sc_agg_h(th0.reshape(2 * N, CW), e0h.reshape(2 * EPAD, CW),
                      src2, dst, zeros)
    prex1 = _sc_agg_x(x, e1x.reshape(EPAD, CW), src, dst, zeros)

    h1, th1 = _mlp_layer0(
        (prex0[0, 0], preh0[0, 0], preh0[0, 1],
         prex0[1, 0], preh0[1, 0], preh0[1, 1]),
        W1_0, b1_0, g_0, bt_0, W2_0, b2_0, W3_0, b3_0)

    preh1 = _sc_agg_h(th1.reshape(2 * N, CW), e1h.reshape(2 * EPAD, CW),
                      src2, dst, zeros)
    out = _mlp_layer1_out(
        (prex1[0, 0], preh1[0, 0], preh1[0, 1],
         prex1[1, 0], preh1[1, 0], preh1[1, 1]),
        h0, h1, W1_1, b1_1, g_1, bt_1, W2_1, b2_1, W3_1, b3_1,
        W_out, b_out)
    return out


# speedup vs baseline: 4.1912x; 1.0235x over previous
import functools

import jax
import jax.numpy as jnp
from jax import lax
from jax.experimental import pallas as pl
from jax.experimental.pallas import tpu as pltpu
from jax.experimental.pallas import tpu_sc as plsc

N = 10000
E = 160000
DIN = 128
H = 256
ED = 16
DOUT = 128
XC = DIN + H
HID2 = 2 * H
CW = 128

NC = 2
NS = 16
NW = NC * NS
LANES = 16
EB = 72
PER_S = E // NW
NRING = 69
TAILR = PER_S - NRING * EB
EPAD = 162000
ROWS_S = 624
ROWS_LAST = N - (NS - 1) * ROWS_S
OFF_LAST = (NS - 1) * ROWS_S

def _dot(a, b):
    return jax.lax.dot_general(a, b, (((1,), (0,)), ((), ())),
                               preferred_element_type=jnp.float32)


def _mlp_in_body(x_ref, w_ref, b_ref, h_ref, t_ref):
    h = _dot(x_ref[...], w_ref[...]) + b_ref[...]
    h_ref[...] = h
    t_ref[0] = h[:, :CW]
    t_ref[1] = h[:, CW:]


def _mlp_in(x, W_in, b_in, blk=1000):
    grid = (N // blk,)
    return pl.pallas_call(
        _mlp_in_body,
        grid=grid,
        in_specs=[
            pl.BlockSpec((blk, DIN), lambda i: (i, 0)),
            pl.BlockSpec((DIN, H), lambda i: (0, 0)),
            pl.BlockSpec((1, H), lambda i: (0, 0)),
        ],
        out_specs=[
            pl.BlockSpec((blk, H), lambda i: (i, 0)),
            pl.BlockSpec((2, blk, CW), lambda i: (0, i, 0)),
        ],
        out_shape=[
            jax.ShapeDtypeStruct((N, H), jnp.float32),
            jax.ShapeDtypeStruct((2, N, CW), jnp.float32),
        ],
    )(x, W_in, b_in.reshape(1, H))


def _edge_feat_body(nch, ea_ref, w_ref, b_ref, out_ref):
    ea = ea_ref[...]
    for ch in range(nch):
        col = ch * CW
        out_ref[ch] = (_dot(ea, w_ref[:, col:col + CW])
                       + b_ref[:, col:col + CW])


def _edge_feat(ea, We, be, blk=2000):
    nch = We.shape[1] // CW
    grid = (EPAD // blk,)
    last = E // blk - 1
    return pl.pallas_call(
        functools.partial(_edge_feat_body, nch),
        grid=grid,
        in_specs=[
            pl.BlockSpec((blk, ED), lambda i: (jnp.minimum(i, last), 0)),
            pl.BlockSpec((ED, nch * CW), lambda i: (0, 0)),
            pl.BlockSpec((1, nch * CW), lambda i: (0, 0)),
        ],
        out_specs=pl.BlockSpec((nch, blk, CW), lambda i: (0, i, 0)),
        out_shape=jax.ShapeDtypeStruct((nch, EPAD, CW), jnp.float32),
    )(ea, We, be.reshape(1, nch * CW))


def _ranged_copy(s, mk_src, mk_dst):
    @pl.when(s < NS - 1)
    def _():
        pltpu.sync_copy(mk_src(s * ROWS_S, ROWS_S), mk_dst(s * ROWS_S, ROWS_S))

    @pl.when(s == NS - 1)
    def _():
        pltpu.sync_copy(mk_src(OFF_LAST, ROWS_LAST), mk_dst(OFF_LAST, ROWS_LAST))


def _sc_body(nph, tbls, e_hbm, idx_hbm, dst_hbm, z_hbm, out_hbm,
             srcb, dstb, ebuf, acc, semE, semG, semD, semS, semI):
    c = lax.axis_index("c")
    s = lax.axis_index("s")
    w = c * NS + s

    for ch in range(nph):
        tbl = tbls[ch]
        idx0 = w * PER_S
        dst0 = w * PER_S

        @pl.when(c == 0)
        def _():
            _ranged_copy(s, lambda o, n: tbl.at[pl.ds(o, n)],
                         lambda o, n: acc.at[pl.ds(o, n)])

        @pl.when(c == 1)
        def _():
            _ranged_copy(s, lambda o, n: z_hbm.at[pl.ds(o, n)],
                         lambda o, n: acc.at[pl.ds(o, n)])

        plsc.subcore_barrier()

        def e_slice(k):
            return e_hbm.at[pl.ds(ch * EPAD + dst0 + k * EB, EB)]

        def fetch_ed(k, b):
            pltpu.async_copy(e_slice(k), ebuf.at[b], semE[b])
            pltpu.async_copy(dst_hbm.at[pl.ds(dst0 + k * EB, EB)],
                             dstb.at[b], semD[b])
            pltpu.async_copy(idx_hbm.at[pl.ds(idx0 + k * EB, EB)],
                             srcb.at[b], semI[b])

        def wait_e(k, b):
            pltpu.make_async_copy(e_slice(k), ebuf.at[b], semE[b]).wait()

        def gather_add(k, b):
            pltpu.make_async_copy(idx_hbm.at[pl.ds(idx0 + k * EB, EB)],
                                  srcb.at[b], semI[b]).wait()
            pltpu.async_copy(tbl.at[srcb.at[b]], ebuf.at[b], semG[b],
                             add=True)

        def wait_gather(k, b):
            pltpu.make_async_copy(tbl.at[srcb.at[b]], ebuf.at[b],
                                  semG[b]).wait()

        def scatter(k, b):
            pltpu.make_async_copy(dst_hbm.at[pl.ds(dst0 + k * EB, EB)],
                                  dstb.at[b], semD[b]).wait()
            pltpu.async_copy(ebuf.at[b], acc.at[dstb.at[b]], semS[b],
                             add=True)

        def wait_scatter(k, b):
            pltpu.make_async_copy(ebuf.at[b], acc.at[dstb.at[b]],
                                  semS[b]).wait()

        def relu(b):
            def row(i, carry2):
                for j in range(CW // LANES):
                    sl = pl.ds(j * LANES, LANES)
                    ebuf[b, i, sl] = jnp.maximum(ebuf[b, i, sl], 0.0)
                return carry2

            lax.fori_loop(0, EB, row, 0, unroll=2)

        fetch_ed(0, 0)
        fetch_ed(1, 1)
        wait_e(0, 0)
        gather_add(0, 0)

        def group(g, carry):
            for j in range(3):
                k = 3 * g + j
                b, b1, b2 = j, (j + 1) % 3, (j + 2) % 3

                @pl.when(k <= NRING - 2)
                def _():
                    wait_e(k + 1, b1)
                    gather_add(k + 1, b1)

                wait_gather(k, b)
                relu(b)
                scatter(k, b)

                @pl.when(k >= 1)
                def _():
                    wait_scatter(k - 1, b2)

                @pl.when(k <= NRING - 3)
                def _():
                    fetch_ed(k + 2, b2)
            return carry

        lax.fori_loop(0, NRING // 3, group, 0)
        wait_scatter(NRING - 1, (NRING - 1) % 3)

        kt = NRING
        fetch_ed(kt, 0)
        wait_e(kt, 0)
        gather_add(kt, 0)
        wait_gather(kt, 0)

        def tail_row(i, carry2):
            for j in range(CW // LANES):
                sl = pl.ds(j * LANES, LANES)
                ebuf[0, i, sl] = jnp.maximum(ebuf[0, i, sl], 0.0)
            return carry2

        lax.fori_loop(0, TAILR, tail_row, 0)

        def zero_row(i, carry2):
            for j in range(CW // LANES):
                ebuf[0, i, pl.ds(j * LANES, LANES)] = jnp.zeros(
                    (LANES,), jnp.float32)
            return carry2

        lax.fori_loop(TAILR, EB, zero_row, 0)
        scatter(kt, 0)
        wait_scatter(kt, 0)

        plsc.subcore_barrier()

        _ranged_copy(s, lambda o, n: acc.at[pl.ds(o, n)],
                     lambda o, n: out_hbm.at[c, ch, pl.ds(o, n)])
        plsc.subcore_barrier()


def _make_sc_agg(nph):
    @functools.partial(
        pl.kernel,
        out_type=jax.ShapeDtypeStruct((NC, nph, N, CW), jnp.float32),
        mesh=plsc.VectorSubcoreMesh(core_axis_name="c", subcore_axis_name="s",
                                    num_cores=NC, num_subcores=NS),
        scratch_types=[
            pltpu.VMEM((3, EB), jnp.int32),
            pltpu.VMEM((3, EB), jnp.int32),
            pltpu.VMEM((3, EB, CW), jnp.float32),
            pltpu.VMEM_SHARED((N, CW), jnp.float32),
        ] + [pltpu.SemaphoreType.DMA] * 15,
    )
    def agg(*args):
        tbls, rest = args[:nph], args[nph:]
        (e_hbm, idx_hbm, dst_hbm, z_hbm, out_hbm,
         srcb, dstb, ebuf, acc) = rest[:9]
        sems = rest[9:]
        _sc_body(nph, tbls, e_hbm, idx_hbm, dst_hbm, z_hbm, out_hbm,
                 srcb, dstb, ebuf, acc,
                 sems[0:3], sems[3:6], sems[6:9], sems[9:12], sems[12:15])

    return agg


_sc_agg_x = _make_sc_agg(1)
_sc_agg_h = _make_sc_agg(2)


def _node_mlp(pre_refs, w1_refs, p):
    h = p['b1']
    for ch in range(3):
        pre = pre_refs[ch][...] + pre_refs[3 + ch][...]
        h = h + _dot(pre, w1_refs[ch][...])
    mu = jnp.mean(h, axis=-1, keepdims=True)
    var = jnp.mean((h - mu) ** 2, axis=-1, keepdims=True)
    h = p['g'] * (h - mu) / jnp.sqrt(var + 1e-5) + p['bt']
    h = jnp.where(h >= 0, h, 0.1 * h)
    h = _dot(h, p['W2']) + p['b2']
    h = jnp.where(h >= 0, h, 0.1 * h)
    return _dot(h, p['W3']) + p['b3']


def _mlp0_body(p00, p01, p02, p10, p11, p12,
               w1a, w1b, w1c, b1, g, bt, w2, b2, w3, b3,
               h_ref, t_ref):
    p = dict(b1=b1[...], g=g[...], bt=bt[...], W2=w2[...], b2=b2[...],
             W3=w3[...], b3=b3[...])
    h = _node_mlp((p00, p01, p02, p10, p11, p12), (w1a, w1b, w1c), p)
    h_ref[...] = h
    t_ref[0] = h[:, :CW]
    t_ref[1] = h[:, CW:]


def _pre_specs(blk):
    return [pl.BlockSpec((blk, CW), lambda i: (i, 0)) for _ in range(6)]


def _wspec(shp):
    return pl.BlockSpec(shp, lambda i: (0, 0))


def _mlp_layer0(pres, W1, b1, g, bt, W2, b2, W3, b3, blk=1000):
    grid = (N // blk,)
    return pl.pallas_call(
        _mlp0_body,
        grid=grid,
        in_specs=_pre_specs(blk) + [
            _wspec((CW, HID2)), _wspec((CW, HID2)), _wspec((CW, HID2)),
            _wspec((1, HID2)), _wspec((1, HID2)), _wspec((1, HID2)),
            _wspec((HID2, HID2)), _wspec((1, HID2)),
            _wspec((HID2, H)), _wspec((1, H)),
        ],
        out_specs=[
            pl.BlockSpec((blk, H), lambda i: (i, 0)),
            pl.BlockSpec((2, blk, CW), lambda i: (0, i, 0)),
        ],
        out_shape=[
            jax.ShapeDtypeStruct((N, H), jnp.float32),
            jax.ShapeDtypeStruct((2, N, CW), jnp.float32),
        ],
    )(*pres,
      W1[:CW], W1[CW:2 * CW], W1[2 * CW:], b1.reshape(1, HID2),
      g.reshape(1, HID2), bt.reshape(1, HID2), W2, b2.reshape(1, HID2),
      W3, b3.reshape(1, H))


def _mlp1_body(p00, p01, p02, p10, p11, p12, h0_ref, h1_ref,
               w1a, w1b, w1c, b1, g, bt, w2, b2, w3, b3,
               wo0, wo1, wo2, bo, out_ref):
    p = dict(b1=b1[...], g=g[...], bt=bt[...], W2=w2[...], b2=b2[...],
             W3=w3[...], b3=b3[...])
    h2 = _node_mlp((p00, p01, p02, p10, p11, p12), (w1a, w1b, w1c), p)
    out_ref[...] = (_dot(h0_ref[...], wo0[...]) + _dot(h1_ref[...], wo1[...])
                    + _dot(h2, wo2[...]) + bo[...])


def _mlp_layer1_out(pres, h0, h1, W1, b1, g, bt, W2, b2, W3, b3,
                    W_out, b_out, blk=1000):
    grid = (N // blk,)
    return pl.pallas_call(
        _mlp1_body,
        grid=grid,
        in_specs=_pre_specs(blk) + [
            pl.BlockSpec((blk, H), lambda i: (i, 0)),
            pl.BlockSpec((blk, H), lambda i: (i, 0)),
            _wspec((CW, HID2)), _wspec((CW, HID2)), _wspec((CW, HID2)),
            _wspec((1, HID2)), _wspec((1, HID2)), _wspec((1, HID2)),
            _wspec((HID2, HID2)), _wspec((1, HID2)),
            _wspec((HID2, H)), _wspec((1, H)),
            _wspec((H, DOUT)), _wspec((H, DOUT)), _wspec((H, DOUT)),
            _wspec((1, DOUT)),
        ],
        out_specs=pl.BlockSpec((blk, DOUT), lambda i: (i, 0)),
        out_shape=jax.ShapeDtypeStruct((N, DOUT), jnp.float32),
    )(*pres,
      h0, h1,
      W1[:CW], W1[CW:2 * CW], W1[2 * CW:], b1.reshape(1, HID2),
      g.reshape(1, HID2), bt.reshape(1, HID2), W2, b2.reshape(1, HID2),
      W3, b3.reshape(1, H),
      W_out[:H], W_out[H:2 * H], W_out[2 * H:], b_out.reshape(1, DOUT))


def kernel(x, edge_index, edge_attr, W_in, b_in,
           We0, be0, W1_0, b1_0, g_0, bt_0, W2_0, b2_0, W3_0, b3_0,
           We1, be1, W1_1, b1_1, g_1, bt_1, W2_1, b2_1, W3_1, b3_1,
           W_out, b_out):
    npad = EPAD - E
    src = jnp.concatenate([edge_index[0], jnp.zeros((npad,), jnp.int32)])
    dst = jnp.concatenate([edge_index[1], jnp.zeros((npad,), jnp.int32)])
    zeros = jnp.zeros((N, CW), jnp.float32)

    h0, th0 = _mlp_in(x, W_in, b_in)

    e0x = _edge_feat(edge_attr, We0[:, :CW], be0[:CW])
    e0h = _edge_feat(edge_attr, We0[:, CW:], be0[CW:])
    e1x = _edge_feat(edge_attr, We1[:, :CW], be1[:CW])
    e1h = _edge_feat(edge_attr, We1[:, CW:], be1[CW:])

    prex0 = _sc_agg_x(x, e0x.reshape(EPAD, CW), src, dst, zeros)
    preh0 = _sc_agg_h(th0[0], th0[1], e0h.reshape(2 * EPAD, CW),
                      src, dst, zeros)
    prex1 = _sc_agg_x(x, e1x.reshape(EPAD, CW), src, dst, zeros)

    h1, th1 = _mlp_layer0(
        (prex0[0, 0], preh0[0, 0], preh0[0, 1],
         prex0[1, 0], preh0[1, 0], preh0[1, 1]),
        W1_0, b1_0, g_0, bt_0, W2_0, b2_0, W3_0, b3_0)

    preh1 = _sc_agg_h(th1[0], th1[1], e1h.reshape(2 * EPAD, CW),
                      src, dst, zeros)
    out = _mlp_layer1_out(
        (prex1[0, 0], preh1[0, 0], preh1[0, 1],
         prex1[1, 0], preh1[1, 0], preh1[1, 1]),
        h0, h1, W1_1, b1_1, g_1, bt_1, W2_1, b2_1, W3_1, b3_1,
        W_out, b_out)
    return out
